# Initial kernel scaffold; baseline (speedup 1.0000x reference)
import functools

import jax
import jax.numpy as jnp
from jax import lax
from jax.experimental import pallas as pl
from jax.experimental.pallas import tpu as pltpu
from jax.experimental.pallas import tpu_sc as plsc

_N = 4096
_D = 128
_L = 64
_E = 65536

_NC = 2
_NS = 16
_NW = _NC * _NS
_CH = 128


def _softplus(x):
    return jnp.maximum(x, 0.0) + jnp.log1p(jnp.exp(-jnp.abs(x)))


def _scalar_tile(vals):
    r = lax.broadcasted_iota(jnp.int32, (8, 128), 0)
    c = lax.broadcasted_iota(jnp.int32, (8, 128), 1)
    out = jnp.zeros((8, 128), jnp.float32)
    for i, v in enumerate(vals):
        out = out + jnp.where((r == 0) & (c == i), v, 0.0)
    return out


def _dinv_block(degp_blk):
    dsum = degp_blk[0] + degp_blk[1]
    deg = 1.0 + dsum[:, 0:1]
    return lax.rsqrt(deg)


def _mlp(x, W1, b1, W2, b2):
    n, d = x.shape
    BN = 512

    def body(xr, w1r, b1r, w2r, b2r, out):
        hb = jnp.maximum(
            jnp.dot(xr[...], w1r[...], preferred_element_type=jnp.float32)
            + b1r[...], 0.0)
        out[...] = (
            jnp.dot(hb, w2r[...], preferred_element_type=jnp.float32)
            + b2r[...])

    return pl.pallas_call(
        body,
        grid=(n // BN,),
        in_specs=[
            pl.BlockSpec((BN, d), lambda i: (i, 0)),
            pl.BlockSpec(W1.shape, lambda i: (0, 0)),
            pl.BlockSpec((1, b1.shape[0]), lambda i: (0, 0)),
            pl.BlockSpec(W2.shape, lambda i: (0, 0)),
            pl.BlockSpec((1, b2.shape[0]), lambda i: (0, 0)),
        ],
        out_specs=pl.BlockSpec((BN, W2.shape[1]), lambda i: (i, 0)),
        out_shape=jax.ShapeDtypeStruct((n, W2.shape[1]), jnp.float32),
    )(x, W1, b1.reshape(1, -1), W2, b2.reshape(1, -1))


def _proj_scale(h, Wg1, degp):
    n, l = h.shape
    BN = 512

    def body(hr, wr, dgr, out):
        dinv = _dinv_block(dgr[...])
        out[...] = jnp.dot(
            hr[...], wr[...], preferred_element_type=jnp.float32) * dinv

    return pl.pallas_call(
        body,
        grid=(n // BN,),
        in_specs=[
            pl.BlockSpec((BN, l), lambda i: (i, 0)),
            pl.BlockSpec(Wg1.shape, lambda i: (0, 0)),
            pl.BlockSpec((2, BN, 16), lambda i: (0, i, 0)),
        ],
        out_specs=pl.BlockSpec((BN, l), lambda i: (i, 0)),
        out_shape=jax.ShapeDtypeStruct((n, l), jnp.float32),
    )(h, Wg1, degp)


def _hid_proj(g1, t1p, Wg2, Wg3, degp):
    n, l = t1p.shape
    BN = 512

    def body(g1r, t1r, dgr, w2r, w3r, out):
        dinv = _dinv_block(dgr[...])
        gs = g1r[0] + g1r[1] + t1r[...]
        hid = jnp.maximum(dinv * gs, 0.0)
        u2 = jnp.dot(hid, w2r[...], preferred_element_type=jnp.float32) * dinv
        u3 = jnp.dot(hid, w3r[...], preferred_element_type=jnp.float32) * dinv
        out[...] = jnp.concatenate([u2, u3], axis=1)

    return pl.pallas_call(
        body,
        grid=(n // BN,),
        in_specs=[
            pl.BlockSpec((2, BN, l), lambda i: (0, i, 0)),
            pl.BlockSpec((BN, l), lambda i: (i, 0)),
            pl.BlockSpec((2, BN, 16), lambda i: (0, i, 0)),
            pl.BlockSpec(Wg2.shape, lambda i: (0, 0)),
            pl.BlockSpec(Wg3.shape, lambda i: (0, 0)),
        ],
        out_specs=pl.BlockSpec((BN, 2 * l), lambda i: (i, 0)),
        out_shape=jax.ShapeDtypeStruct((n, 2 * l), jnp.float32),
    )(g1, t1p, degp, Wg2, Wg3)


def _mu_kld(g2, U, degp):
    n, d2 = U.shape
    l = d2 // 2
    BN = 512
    nblk = n // BN

    def body(g2r, ur, dgr, mu_out, sc_out, acc):
        i = pl.program_id(0)

        @pl.when(i == 0)
        def _():
            acc[0] = 0.0
            acc[1] = 0.0
            acc[2] = 0.0

        dinv = _dinv_block(dgr[...])
        v = dinv * (g2r[0] + g2r[1] + ur[...])
        mu = v[:, :l]
        lv = v[:, l:]
        mu_out[...] = mu
        kt = jnp.sum(1.0 + 2.0 * lv - mu * mu - jnp.exp(2.0 * lv))
        q = jnp.sum(mu * mu, axis=1)
        acc[0] = acc[0] + kt
        acc[1] = acc[1] + jnp.sum(_softplus(q))
        acc[2] = acc[2] + jnp.sum(q)

        @pl.when(i == nblk - 1)
        def _():
            sc_out[...] = _scalar_tile([acc[0], acc[1], acc[2]])

    return pl.pallas_call(
        body,
        grid=(nblk,),
        in_specs=[
            pl.BlockSpec((2, BN, d2), lambda i: (0, i, 0)),
            pl.BlockSpec((BN, d2), lambda i: (i, 0)),
            pl.BlockSpec((2, BN, 16), lambda i: (0, i, 0)),
        ],
        out_specs=[
            pl.BlockSpec((BN, l), lambda i: (i, 0)),
            pl.BlockSpec((8, 128), lambda i: (0, 0)),
        ],
        out_shape=[
            jax.ShapeDtypeStruct((n, l), jnp.float32),
            jax.ShapeDtypeStruct((8, 128), jnp.float32),
        ],
        scratch_shapes=[pltpu.SMEM((4,), jnp.float32)],
    )(g2, U, degp)


def _dense_loss(mu):
    n, l = mu.shape
    BN = 512
    nblk = n // BN

    def body(ar, br, out, acc):
        i = pl.program_id(0)
        j = pl.program_id(1)

        @pl.when((i == 0) & (j == 0))
        def _():
            acc[0] = 0.0

        p = lax.dot_general(
            ar[...], br[...], (((1,), (1,)), ((), ())),
            preferred_element_type=jnp.float32)
        acc[0] = acc[0] + jnp.sum(_softplus(p))

        @pl.when((i == nblk - 1) & (j == nblk - 1))
        def _():
            out[...] = _scalar_tile([acc[0]])

    return pl.pallas_call(
        body,
        grid=(nblk, nblk),
        in_specs=[
            pl.BlockSpec((BN, l), lambda i, j: (i, 0)),
            pl.BlockSpec((BN, l), lambda i, j: (j, 0)),
        ],
        out_specs=pl.BlockSpec((8, 128), lambda i, j: (0, 0)),
        out_shape=jax.ShapeDtypeStruct((8, 128), jnp.float32),
        scratch_shapes=[pltpu.SMEM((1,), jnp.float32)],
    )(mu, mu)


def _edge_loss(zs, zd, src, dst):
    e, l = zs.shape
    BE = 4096
    nblk = e // BE

    def body(zsr, zdr, sr, dr, out, acc):
        i = pl.program_id(0)

        @pl.when(i == 0)
        def _():
            acc[0] = 0.0
            acc[1] = 0.0

        p = jnp.sum(zsr[...] * zdr[...], axis=1)
        m = (sr[...] != dr[...]).astype(jnp.float32)
        acc[0] = acc[0] + jnp.sum(m * _softplus(p))
        acc[1] = acc[1] + jnp.sum(m * p)

        @pl.when(i == nblk - 1)
        def _():
            out[...] = _scalar_tile([acc[0], acc[1]])

    return pl.pallas_call(
        body,
        grid=(nblk,),
        in_specs=[
            pl.BlockSpec((BE, l), lambda i: (i, 0)),
            pl.BlockSpec((BE, l), lambda i: (i, 0)),
            pl.BlockSpec((BE,), lambda i: (i,)),
            pl.BlockSpec((BE,), lambda i: (i,)),
        ],
        out_specs=pl.BlockSpec((8, 128), lambda i: (0, 0)),
        out_shape=jax.ShapeDtypeStruct((8, 128), jnp.float32),
        scratch_shapes=[pltpu.SMEM((2,), jnp.float32)],
    )(zs, zd, src, dst)


def _sc_mesh():
    return plsc.VectorSubcoreMesh(core_axis_name="c", subcore_axis_name="s")


def _zero_rows(rows, d):
    z = jnp.zeros((16,), jnp.float32)

    def zb(i, carry):
        for j in range(d // 16):
            rows[i, pl.ds(j * 16, 16)] = z
        return carry

    lax.fori_loop(0, _CH, zb, 0)


def _sc_degree(src):
    rows_per_tile = _N // _NS
    epw = _E // _NW

    @functools.partial(
        pl.kernel,
        out_type=jax.ShapeDtypeStruct((_NC, _N, 16), jnp.float32),
        mesh=_sc_mesh(),
        scratch_types=[
            pltpu.VMEM((_CH,), jnp.int32),
            pltpu.VMEM((_CH, 16), jnp.float32),
            pltpu.VMEM_SHARED((_N, 16), jnp.float32),
            pltpu.SemaphoreType.DMA,
        ],
    )
    def k(src_hbm, out_hbm, idx_v, ones_v, acc_sh, sem):
        cid = lax.axis_index("c")
        sid = lax.axis_index("s")
        _zero_rows(ones_v, 16)
        for r in range(rows_per_tile // _CH):
            pltpu.sync_copy(
                ones_v, acc_sh.at[pl.ds(sid * rows_per_tile + r * _CH, _CH)])
        one = jnp.ones((16,), jnp.float32)

        def ob(i, carry):
            ones_v[i, :] = one
            return carry

        lax.fori_loop(0, _CH, ob, 0)
        plsc.subcore_barrier()
        base0 = (cid * _NS + sid) * epw
        for kk in range(epw // _CH):
            pltpu.sync_copy(src_hbm.at[pl.ds(base0 + kk * _CH, _CH)], idx_v)
            pltpu.sync_copy(ones_v, acc_sh.at[idx_v], add=True)
        plsc.subcore_barrier()
        pltpu.sync_copy(
            acc_sh.at[pl.ds(sid * rows_per_tile, rows_per_tile)],
            out_hbm.at[cid, pl.ds(sid * rows_per_tile, rows_per_tile)])

    return k(src)


def _sc_gather_scatter(s, src, dst):
    n, d = s.shape
    rows_per_tile = n // _NS
    epw = _E // _NW

    @functools.partial(
        pl.kernel,
        out_type=jax.ShapeDtypeStruct((_NC, n, d), jnp.float32),
        mesh=_sc_mesh(),
        scratch_types=[
            pltpu.VMEM((_CH,), jnp.int32),
            pltpu.VMEM((_CH,), jnp.int32),
            pltpu.VMEM((_CH, d), jnp.float32),
            pltpu.VMEM_SHARED((n, d), jnp.float32),
            pltpu.SemaphoreType.DMA,
        ],
    )
    def k(s_hbm, src_hbm, dst_hbm, out_hbm, sidx, didx, rows, acc_sh, sem):
        cid = lax.axis_index("c")
        sid = lax.axis_index("s")
        _zero_rows(rows, d)
        for r in range(rows_per_tile // _CH):
            pltpu.sync_copy(
                rows, acc_sh.at[pl.ds(sid * rows_per_tile + r * _CH, _CH)])
        plsc.subcore_barrier()
        base0 = (cid * _NS + sid) * epw
        for kk in range(epw // _CH):
            base = base0 + kk * _CH
            pltpu.sync_copy(dst_hbm.at[pl.ds(base, _CH)], didx)
            pltpu.async_copy(s_hbm.at[didx], rows, sem).wait()
            pltpu.sync_copy(src_hbm.at[pl.ds(base, _CH)], sidx)
            pltpu.sync_copy(rows, acc_sh.at[sidx], add=True)
        plsc.subcore_barrier()
        pltpu.sync_copy(
            acc_sh.at[pl.ds(sid * rows_per_tile, rows_per_tile)],
            out_hbm.at[cid, pl.ds(sid * rows_per_tile, rows_per_tile)])

    return k(s, src, dst)


def _sc_edge_gather(mu, src, dst):
    n, d = mu.shape
    epw = _E // _NW

    @functools.partial(
        pl.kernel,
        out_type=(
            jax.ShapeDtypeStruct((_E, d), jnp.float32),
            jax.ShapeDtypeStruct((_E, d), jnp.float32),
        ),
        mesh=_sc_mesh(),
        scratch_types=[
            pltpu.VMEM((_CH,), jnp.int32),
            pltpu.VMEM((_CH, d), jnp.float32),
            pltpu.SemaphoreType.DMA,
        ],
    )
    def k(mu_hbm, src_hbm, dst_hbm, zs_hbm, zd_hbm, idx, rows, sem):
        cid = lax.axis_index("c")
        sid = lax.axis_index("s")
        base0 = (cid * _NS + sid) * epw
        for kk in range(epw // _CH):
            base = base0 + kk * _CH
            pltpu.sync_copy(src_hbm.at[pl.ds(base, _CH)], idx)
            pltpu.async_copy(mu_hbm.at[idx], rows, sem).wait()
            pltpu.sync_copy(rows, zs_hbm.at[pl.ds(base, _CH)])
            pltpu.sync_copy(dst_hbm.at[pl.ds(base, _CH)], idx)
            pltpu.async_copy(mu_hbm.at[idx], rows, sem).wait()
            pltpu.sync_copy(rows, zd_hbm.at[pl.ds(base, _CH)])

    return k(mu, src, dst)


def kernel(x, edge_index, W1, b1, W2, b2, Wg1, Wg2, Wg3):
    n = x.shape[0]
    e = edge_index.shape[1]
    src = edge_index[0].astype(jnp.int32)
    dst = edge_index[1].astype(jnp.int32)

    h = _mlp(x, W1, b1, W2, b2)
    degp = _sc_degree(src)

    t1p = _proj_scale(h, Wg1, degp)
    g1 = _sc_gather_scatter(t1p, src, dst)
    U = _hid_proj(g1, t1p, Wg2, Wg3, degp)
    g2 = _sc_gather_scatter(U, src, dst)
    mu, sc1 = _mu_kld(g2, U, degp)

    zs, zd = _sc_edge_gather(mu, src, dst)
    sc2 = _edge_loss(zs, zd, src, dst)
    sall = _dense_loss(mu)

    kld_sum = sc1[0, 0]
    diag_sp = sc1[0, 1]
    diag_p = sc1[0, 2]
    edge_sp = sc2[0, 0]
    edge_p = sc2[0, 1]
    all_sp = sall[0, 0]

    total = float(n) * float(n)
    s_edges = float(e)
    pos_weight = (total - s_edges) / s_edges
    norm = total / ((total - s_edges) * 2.0)

    bce = (all_sp
           + (pos_weight - 1.0) * (edge_sp + diag_sp)
           - pos_weight * (edge_p + diag_p)) / total
    kld = (-0.5) * kld_sum / total
    gae_loss = norm * bce + kld
    return h, gae_loss


# speedup vs baseline: 130.5412x; 130.5412x over previous
import functools

import jax
import jax.numpy as jnp
from jax import lax
from jax.experimental import pallas as pl
from jax.experimental.pallas import tpu as pltpu
from jax.experimental.pallas import tpu_sc as plsc

_N = 4096
_D = 128
_L = 64
_E = 65536

_NC = 2
_NS = 16
_NW = _NC * _NS
_CH = 128


def _softplus(x):
    return jnp.maximum(x, 0.0) + jnp.log1p(jnp.exp(-jnp.abs(x)))


def _scalar_tile(vals):
    r = lax.broadcasted_iota(jnp.int32, (8, 128), 0)
    c = lax.broadcasted_iota(jnp.int32, (8, 128), 1)
    out = jnp.zeros((8, 128), jnp.float32)
    for i, v in enumerate(vals):
        out = out + jnp.where((r == 0) & (c == i), v, 0.0)
    return out


def _dinv_block(degp_blk):
    deg = 1.0 + degp_blk[0][:, 0:1] + degp_blk[1][:, 0:1]
    return lax.rsqrt(deg)


def _mlp(x, W1, b1, W2, b2):
    n, d = x.shape
    BN = 512

    def body(xr, w1r, b1r, w2r, b2r, out):
        hb = jnp.maximum(
            jnp.dot(xr[...], w1r[...], preferred_element_type=jnp.float32, precision=lax.Precision.HIGHEST)
            + b1r[...], 0.0)
        out[...] = (
            jnp.dot(hb, w2r[...], preferred_element_type=jnp.float32, precision=lax.Precision.HIGHEST)
            + b2r[...])

    return pl.pallas_call(
        body,
        grid=(n // BN,),
        in_specs=[
            pl.BlockSpec((BN, d), lambda i: (i, 0)),
            pl.BlockSpec(W1.shape, lambda i: (0, 0)),
            pl.BlockSpec((1, b1.shape[0]), lambda i: (0, 0)),
            pl.BlockSpec(W2.shape, lambda i: (0, 0)),
            pl.BlockSpec((1, b2.shape[0]), lambda i: (0, 0)),
        ],
        out_specs=pl.BlockSpec((BN, W2.shape[1]), lambda i: (i, 0)),
        out_shape=jax.ShapeDtypeStruct((n, W2.shape[1]), jnp.float32),
    )(x, W1, b1.reshape(1, -1), W2, b2.reshape(1, -1))


def _proj_scale(h, Wg1, degp):
    n, l = h.shape
    BN = 512

    def body(hr, wr, dgr, out, dv_out):
        dinv = _dinv_block(dgr[...])
        t = jnp.dot(hr[...], wr[...], preferred_element_type=jnp.float32, precision=lax.Precision.HIGHEST) * dinv
        out[...] = jnp.concatenate(
            [t, jnp.zeros((BN, 128 - l), jnp.float32)], axis=1)
        dv_out[...] = jnp.broadcast_to(dinv, (BN, 16))

    return pl.pallas_call(
        body,
        grid=(n // BN,),
        in_specs=[
            pl.BlockSpec((BN, l), lambda i: (i, 0)),
            pl.BlockSpec(Wg1.shape, lambda i: (0, 0)),
            pl.BlockSpec((2, BN, 128), lambda i: (0, i, 0)),
        ],
        out_specs=[
            pl.BlockSpec((BN, 128), lambda i: (i, 0)),
            pl.BlockSpec((BN, 16), lambda i: (i, 0)),
        ],
        out_shape=[
            jax.ShapeDtypeStruct((n, 128), jnp.float32),
            jax.ShapeDtypeStruct((n, 16), jnp.float32),
        ],
    )(h, Wg1, degp)


def _hid_proj(g1, t1p, Wg2, Wg3, dinv16):
    n, _ = t1p.shape
    l = Wg2.shape[0]
    BN = 512

    def body(g1r, t1r, dvr, w2r, w3r, out):
        dinv = dvr[...][:, 0:1]
        gs = g1r[0] + g1r[1] + t1r[...]
        hid = jnp.maximum(dinv * gs[:, :l], 0.0)
        u2 = jnp.dot(hid, w2r[...], preferred_element_type=jnp.float32, precision=lax.Precision.HIGHEST) * dinv
        u3 = jnp.dot(hid, w3r[...], preferred_element_type=jnp.float32, precision=lax.Precision.HIGHEST) * dinv
        out[...] = jnp.concatenate([u2, u3], axis=1)

    return pl.pallas_call(
        body,
        grid=(n // BN,),
        in_specs=[
            pl.BlockSpec((2, BN, 128), lambda i: (0, i, 0)),
            pl.BlockSpec((BN, 128), lambda i: (i, 0)),
            pl.BlockSpec((BN, 16), lambda i: (i, 0)),
            pl.BlockSpec(Wg2.shape, lambda i: (0, 0)),
            pl.BlockSpec(Wg3.shape, lambda i: (0, 0)),
        ],
        out_specs=pl.BlockSpec((BN, 2 * l), lambda i: (i, 0)),
        out_shape=jax.ShapeDtypeStruct((n, 2 * l), jnp.float32),
    )(g1, t1p, dinv16, Wg2, Wg3)


def _mu_kld(g2, U, dinv16):
    n, d2 = U.shape
    l = d2 // 2
    BN = 512
    nblk = n // BN

    def body(g2r, ur, dvr, mu_out, sc_out, acc):
        i = pl.program_id(0)

        @pl.when(i == 0)
        def _():
            acc[0] = 0.0
            acc[1] = 0.0
            acc[2] = 0.0

        dinv = dvr[...][:, 0:1]
        v = dinv * (g2r[0] + g2r[1] + ur[...])
        mu = v[:, :l]
        lv = v[:, l:]
        mu_out[...] = jnp.concatenate(
            [mu, jnp.zeros((BN, d2 - l), jnp.float32)], axis=1)
        kt = jnp.sum(1.0 + 2.0 * lv - mu * mu - jnp.exp(2.0 * lv))
        q = jnp.sum(mu * mu, axis=1)
        acc[0] = acc[0] + kt
        acc[1] = acc[1] + jnp.sum(_softplus(q))
        acc[2] = acc[2] + jnp.sum(q)

        @pl.when(i == nblk - 1)
        def _():
            sc_out[...] = _scalar_tile([acc[0], acc[1], acc[2]])

    return pl.pallas_call(
        body,
        grid=(nblk,),
        in_specs=[
            pl.BlockSpec((2, BN, d2), lambda i: (0, i, 0)),
            pl.BlockSpec((BN, d2), lambda i: (i, 0)),
            pl.BlockSpec((BN, 16), lambda i: (i, 0)),
        ],
        out_specs=[
            pl.BlockSpec((BN, d2), lambda i: (i, 0)),
            pl.BlockSpec((8, 128), lambda i: (0, 0)),
        ],
        out_shape=[
            jax.ShapeDtypeStruct((n, d2), jnp.float32),
            jax.ShapeDtypeStruct((8, 128), jnp.float32),
        ],
        scratch_shapes=[pltpu.SMEM((4,), jnp.float32)],
    )(g2, U, dinv16)


def _dense_loss(mu):
    n, l = mu.shape
    BN = 512
    nblk = n // BN

    def body(ar, br, out, acc):
        i = pl.program_id(0)
        j = pl.program_id(1)

        @pl.when((i == 0) & (j == 0))
        def _():
            acc[0] = 0.0

        p = lax.dot_general(
            ar[...], br[...], (((1,), (1,)), ((), ())),
            preferred_element_type=jnp.float32,
            precision=lax.Precision.HIGHEST)
        acc[0] = acc[0] + jnp.sum(_softplus(p))

        @pl.when((i == nblk - 1) & (j == nblk - 1))
        def _():
            out[...] = _scalar_tile([acc[0]])

    return pl.pallas_call(
        body,
        grid=(nblk, nblk),
        in_specs=[
            pl.BlockSpec((BN, l), lambda i, j: (i, 0)),
            pl.BlockSpec((BN, l), lambda i, j: (j, 0)),
        ],
        out_specs=pl.BlockSpec((8, 128), lambda i, j: (0, 0)),
        out_shape=jax.ShapeDtypeStruct((8, 128), jnp.float32),
        scratch_shapes=[pltpu.SMEM((1,), jnp.float32)],
    )(mu, mu)


def _edge_loss(zs, zd, src, dst):
    e, l = zs.shape
    BE = 4096
    nblk = e // BE

    def body(zsr, zdr, sr, dr, out, acc):
        i = pl.program_id(0)

        @pl.when(i == 0)
        def _():
            acc[0] = 0.0
            acc[1] = 0.0

        p = jnp.sum(zsr[...] * zdr[...], axis=1)
        m = (sr[...] != dr[...]).astype(jnp.float32)
        acc[0] = acc[0] + jnp.sum(m * _softplus(p))
        acc[1] = acc[1] + jnp.sum(m * p)

        @pl.when(i == nblk - 1)
        def _():
            out[...] = _scalar_tile([acc[0], acc[1]])

    return pl.pallas_call(
        body,
        grid=(nblk,),
        in_specs=[
            pl.BlockSpec((BE, l), lambda i: (i, 0)),
            pl.BlockSpec((BE, l), lambda i: (i, 0)),
            pl.BlockSpec((BE,), lambda i: (i,)),
            pl.BlockSpec((BE,), lambda i: (i,)),
        ],
        out_specs=pl.BlockSpec((8, 128), lambda i: (0, 0)),
        out_shape=jax.ShapeDtypeStruct((8, 128), jnp.float32),
        scratch_shapes=[pltpu.SMEM((2,), jnp.float32)],
    )(zs, zd, src, dst)


def _sc_mesh():
    return plsc.VectorSubcoreMesh(core_axis_name="c", subcore_axis_name="s")


def _zero_rows(rows, d):
    z = jnp.zeros((16,), jnp.float32)

    def zb(i, carry):
        for j in range(d // 16):
            rows[i, pl.ds(j * 16, 16)] = z
        return carry

    lax.fori_loop(jnp.int32(0), jnp.int32(_CH), zb, jnp.int32(0))


def _sc_degree(src):
    rows_per_tile = _N // _NS
    epw = _E // _NW

    @functools.partial(
        pl.kernel,
        out_type=jax.ShapeDtypeStruct((_NC, _N, 128), jnp.float32),
        mesh=_sc_mesh(),
        scratch_types=[
            pltpu.VMEM((_CH,), jnp.int32),
            pltpu.VMEM((_CH, 128), jnp.float32),
            pltpu.VMEM_SHARED((_N, 128), jnp.float32),
            pltpu.SemaphoreType.DMA,
        ],
    )
    def k(src_hbm, out_hbm, idx_v, ones_v, acc_sh, sem):
        cid = lax.axis_index("c")
        sid = lax.axis_index("s")
        _zero_rows(ones_v, 128)
        for r in range(rows_per_tile // _CH):
            pltpu.sync_copy(
                ones_v, acc_sh.at[pl.ds(sid * rows_per_tile + r * _CH, _CH)])
        one = jnp.ones((16,), jnp.float32)

        def ob(i, carry):
            for j in range(128 // 16):
                ones_v[i, pl.ds(j * 16, 16)] = one
            return carry

        lax.fori_loop(jnp.int32(0), jnp.int32(_CH), ob, jnp.int32(0))
        plsc.subcore_barrier()
        base0 = (cid * _NS + sid) * epw
        for kk in range(epw // _CH):
            pltpu.sync_copy(src_hbm.at[pl.ds(base0 + kk * _CH, _CH)], idx_v)
            pltpu.sync_copy(ones_v, acc_sh.at[idx_v], add=True)
        plsc.subcore_barrier()
        pltpu.sync_copy(
            acc_sh.at[pl.ds(sid * rows_per_tile, rows_per_tile)],
            out_hbm.at[cid, pl.ds(sid * rows_per_tile, rows_per_tile)])

    return k(src)


def _sc_gather_scatter(s, src, dst):
    n, d = s.shape
    rows_per_tile = n // _NS
    epw = _E // _NW

    @functools.partial(
        pl.kernel,
        out_type=jax.ShapeDtypeStruct((_NC, n, d), jnp.float32),
        mesh=_sc_mesh(),
        scratch_types=[
            pltpu.VMEM((_CH,), jnp.int32),
            pltpu.VMEM((_CH,), jnp.int32),
            pltpu.VMEM((_CH, d), jnp.float32),
            pltpu.VMEM_SHARED((n, d), jnp.float32),
            pltpu.SemaphoreType.DMA,
        ],
    )
    def k(s_hbm, src_hbm, dst_hbm, out_hbm, sidx, didx, rows, acc_sh, sem):
        cid = lax.axis_index("c")
        sid = lax.axis_index("s")
        _zero_rows(rows, d)
        for r in range(rows_per_tile // _CH):
            pltpu.sync_copy(
                rows, acc_sh.at[pl.ds(sid * rows_per_tile + r * _CH, _CH)])
        plsc.subcore_barrier()
        base0 = (cid * _NS + sid) * epw
        for kk in range(epw // _CH):
            base = base0 + kk * _CH
            pltpu.sync_copy(dst_hbm.at[pl.ds(base, _CH)], didx)
            pltpu.async_copy(s_hbm.at[didx], rows, sem).wait()
            pltpu.sync_copy(src_hbm.at[pl.ds(base, _CH)], sidx)
            pltpu.sync_copy(rows, acc_sh.at[sidx], add=True)
        plsc.subcore_barrier()
        pltpu.sync_copy(
            acc_sh.at[pl.ds(sid * rows_per_tile, rows_per_tile)],
            out_hbm.at[cid, pl.ds(sid * rows_per_tile, rows_per_tile)])

    return k(s, src, dst)


def _sc_edge_gather(mu, src, dst):
    n, d = mu.shape
    epw = _E // _NW

    @functools.partial(
        pl.kernel,
        out_type=(
            jax.ShapeDtypeStruct((_E, d), jnp.float32),
            jax.ShapeDtypeStruct((_E, d), jnp.float32),
        ),
        mesh=_sc_mesh(),
        scratch_types=[
            pltpu.VMEM((_CH,), jnp.int32),
            pltpu.VMEM((_CH, d), jnp.float32),
            pltpu.SemaphoreType.DMA,
        ],
    )
    def k(mu_hbm, src_hbm, dst_hbm, zs_hbm, zd_hbm, idx, rows, sem):
        cid = lax.axis_index("c")
        sid = lax.axis_index("s")
        base0 = (cid * _NS + sid) * epw
        for kk in range(epw // _CH):
            base = base0 + kk * _CH
            pltpu.sync_copy(src_hbm.at[pl.ds(base, _CH)], idx)
            pltpu.async_copy(mu_hbm.at[idx], rows, sem).wait()
            pltpu.sync_copy(rows, zs_hbm.at[pl.ds(base, _CH)])
            pltpu.sync_copy(dst_hbm.at[pl.ds(base, _CH)], idx)
            pltpu.async_copy(mu_hbm.at[idx], rows, sem).wait()
            pltpu.sync_copy(rows, zd_hbm.at[pl.ds(base, _CH)])

    return k(mu, src, dst)


def kernel(x, edge_index, W1, b1, W2, b2, Wg1, Wg2, Wg3):
    with jax.enable_x64(False):
        h, gae_loss = _run(x, edge_index, W1, b1, W2, b2, Wg1, Wg2, Wg3)
    return h.astype(jnp.float64), gae_loss.astype(jnp.float64)


def _run(x, edge_index, W1, b1, W2, b2, Wg1, Wg2, Wg3):
    n = x.shape[0]
    e = edge_index.shape[1]
    src = edge_index[0].astype(jnp.int32)
    dst = edge_index[1].astype(jnp.int32)
    x = x.astype(jnp.float32)
    W1 = W1.astype(jnp.float32)
    b1 = b1.astype(jnp.float32)
    W2 = W2.astype(jnp.float32)
    b2 = b2.astype(jnp.float32)
    Wg1 = Wg1.astype(jnp.float32)
    Wg2 = Wg2.astype(jnp.float32)
    Wg3 = Wg3.astype(jnp.float32)

    h = _mlp(x, W1, b1, W2, b2)
    degp = _sc_degree(src)

    t1p, dinv16 = _proj_scale(h, Wg1, degp)
    g1 = _sc_gather_scatter(t1p, src, dst)
    U = _hid_proj(g1, t1p, Wg2, Wg3, dinv16)
    g2 = _sc_gather_scatter(U, src, dst)
    mu, sc1 = _mu_kld(g2, U, dinv16)

    zs, zd = _sc_edge_gather(mu, src, dst)
    sc2 = _edge_loss(zs, zd, src, dst)
    sall = _dense_loss(mu)

    kld_sum = sc1[0, 0]
    diag_sp = sc1[0, 1]
    diag_p = sc1[0, 2]
    edge_sp = sc2[0, 0]
    edge_p = sc2[0, 1]
    all_sp = sall[0, 0]

    total = float(n) * float(n)
    s_edges = float(e)
    pos_weight = (total - s_edges) / s_edges
    norm = total / ((total - s_edges) * 2.0)

    bce = (all_sp
           + (pos_weight - 1.0) * (edge_sp + diag_sp)
           - pos_weight * (edge_p + diag_p)) / total
    kld = (-0.5) * kld_sum / total
    gae_loss = norm * bce + kld
    return h, gae_loss


# speedup vs baseline: 152.4330x; 1.1677x over previous
import functools

import jax
import jax.numpy as jnp
from jax import lax
from jax.experimental import pallas as pl
from jax.experimental.pallas import tpu as pltpu
from jax.experimental.pallas import tpu_sc as plsc

_N = 4096
_D = 128
_L = 64
_E = 65536

_NC = 2
_NS = 16
_NW = _NC * _NS
_CH = 128


def _softplus(x):
    return jnp.maximum(x, 0.0) + jnp.log1p(jnp.exp(-jnp.abs(x)))


def _scalar_tile(vals):
    r = lax.broadcasted_iota(jnp.int32, (8, 128), 0)
    c = lax.broadcasted_iota(jnp.int32, (8, 128), 1)
    out = jnp.zeros((8, 128), jnp.float32)
    for i, v in enumerate(vals):
        out = out + jnp.where((r == 0) & (c == i), v, 0.0)
    return out


def _dinv_block(degp_blk):
    deg = 1.0 + degp_blk[0][:, 0:1] + degp_blk[1][:, 0:1]
    return lax.rsqrt(deg)


def _mlp(x, W1, b1, W2, b2):
    n, d = x.shape
    BN = 512

    def body(xr, w1r, b1r, w2r, b2r, out):
        hb = jnp.maximum(
            jnp.dot(xr[...], w1r[...], preferred_element_type=jnp.float32, precision=lax.Precision.HIGHEST)
            + b1r[...], 0.0)
        out[...] = (
            jnp.dot(hb, w2r[...], preferred_element_type=jnp.float32, precision=lax.Precision.HIGHEST)
            + b2r[...])

    return pl.pallas_call(
        body,
        grid=(n // BN,),
        in_specs=[
            pl.BlockSpec((BN, d), lambda i: (i, 0)),
            pl.BlockSpec(W1.shape, lambda i: (0, 0)),
            pl.BlockSpec((1, b1.shape[0]), lambda i: (0, 0)),
            pl.BlockSpec(W2.shape, lambda i: (0, 0)),
            pl.BlockSpec((1, b2.shape[0]), lambda i: (0, 0)),
        ],
        out_specs=pl.BlockSpec((BN, W2.shape[1]), lambda i: (i, 0)),
        out_shape=jax.ShapeDtypeStruct((n, W2.shape[1]), jnp.float32),
    )(x, W1, b1.reshape(1, -1), W2, b2.reshape(1, -1))


def _proj_scale(h, Wg1, degp):
    n, l = h.shape
    BN = 512

    def body(hr, wr, dgr, out, dv_out):
        dinv = _dinv_block(dgr[...])
        t = jnp.dot(hr[...], wr[...], preferred_element_type=jnp.float32, precision=lax.Precision.HIGHEST) * dinv
        out[...] = jnp.concatenate(
            [t, jnp.zeros((BN, 128 - l), jnp.float32)], axis=1)
        dv_out[...] = jnp.broadcast_to(dinv, (BN, 16))

    return pl.pallas_call(
        body,
        grid=(n // BN,),
        in_specs=[
            pl.BlockSpec((BN, l), lambda i: (i, 0)),
            pl.BlockSpec(Wg1.shape, lambda i: (0, 0)),
            pl.BlockSpec((2, BN, 128), lambda i: (0, i, 0)),
        ],
        out_specs=[
            pl.BlockSpec((BN, 128), lambda i: (i, 0)),
            pl.BlockSpec((BN, 16), lambda i: (i, 0)),
        ],
        out_shape=[
            jax.ShapeDtypeStruct((n, 128), jnp.float32),
            jax.ShapeDtypeStruct((n, 16), jnp.float32),
        ],
    )(h, Wg1, degp)


def _hid_proj(g1, t1p, Wg2, Wg3, dinv16):
    n, _ = t1p.shape
    l = Wg2.shape[0]
    BN = 512

    def body(g1r, t1r, dvr, w2r, w3r, out):
        dinv = dvr[...][:, 0:1]
        gs = g1r[0] + g1r[1] + t1r[...]
        hid = jnp.maximum(dinv * gs[:, :l], 0.0)
        u2 = jnp.dot(hid, w2r[...], preferred_element_type=jnp.float32, precision=lax.Precision.HIGHEST) * dinv
        u3 = jnp.dot(hid, w3r[...], preferred_element_type=jnp.float32, precision=lax.Precision.HIGHEST) * dinv
        out[...] = jnp.concatenate([u2, u3], axis=1)

    return pl.pallas_call(
        body,
        grid=(n // BN,),
        in_specs=[
            pl.BlockSpec((2, BN, 128), lambda i: (0, i, 0)),
            pl.BlockSpec((BN, 128), lambda i: (i, 0)),
            pl.BlockSpec((BN, 16), lambda i: (i, 0)),
            pl.BlockSpec(Wg2.shape, lambda i: (0, 0)),
            pl.BlockSpec(Wg3.shape, lambda i: (0, 0)),
        ],
        out_specs=pl.BlockSpec((BN, 2 * l), lambda i: (i, 0)),
        out_shape=jax.ShapeDtypeStruct((n, 2 * l), jnp.float32),
    )(g1, t1p, dinv16, Wg2, Wg3)


def _mu_kld(g2, U, dinv16):
    n, d2 = U.shape
    l = d2 // 2
    BN = 512
    nblk = n // BN

    def body(g2r, ur, dvr, mu_out, sc_out, acc):
        i = pl.program_id(0)

        @pl.when(i == 0)
        def _():
            acc[0] = 0.0
            acc[1] = 0.0
            acc[2] = 0.0

        dinv = dvr[...][:, 0:1]
        v = dinv * (g2r[0] + g2r[1] + ur[...])
        mu = v[:, :l]
        lv = v[:, l:]
        mu_out[...] = jnp.concatenate(
            [mu, jnp.zeros((BN, d2 - l), jnp.float32)], axis=1)
        kt = jnp.sum(1.0 + 2.0 * lv - mu * mu - jnp.exp(2.0 * lv))
        q = jnp.sum(mu * mu, axis=1)
        acc[0] = acc[0] + kt
        acc[1] = acc[1] + jnp.sum(_softplus(q))
        acc[2] = acc[2] + jnp.sum(q)

        @pl.when(i == nblk - 1)
        def _():
            sc_out[...] = _scalar_tile([acc[0], acc[1], acc[2]])

    return pl.pallas_call(
        body,
        grid=(nblk,),
        in_specs=[
            pl.BlockSpec((2, BN, d2), lambda i: (0, i, 0)),
            pl.BlockSpec((BN, d2), lambda i: (i, 0)),
            pl.BlockSpec((BN, 16), lambda i: (i, 0)),
        ],
        out_specs=[
            pl.BlockSpec((BN, d2), lambda i: (i, 0)),
            pl.BlockSpec((8, 128), lambda i: (0, 0)),
        ],
        out_shape=[
            jax.ShapeDtypeStruct((n, d2), jnp.float32),
            jax.ShapeDtypeStruct((8, 128), jnp.float32),
        ],
        scratch_shapes=[pltpu.SMEM((4,), jnp.float32)],
    )(g2, U, dinv16)


def _dense_loss(mu):
    n, l = mu.shape
    BN = 512
    nblk = n // BN

    def body(ar, br, out, acc):
        i = pl.program_id(0)
        j = pl.program_id(1)

        @pl.when((i == 0) & (j == 0))
        def _():
            acc[0] = 0.0

        p = lax.dot_general(
            ar[...][:, :_L], br[...][:, :_L], (((1,), (1,)), ((), ())),
            preferred_element_type=jnp.float32,
            precision=lax.Precision.HIGHEST)
        acc[0] = acc[0] + jnp.sum(_softplus(p))

        @pl.when((i == nblk - 1) & (j == nblk - 1))
        def _():
            out[...] = _scalar_tile([acc[0]])

    return pl.pallas_call(
        body,
        grid=(nblk, nblk),
        in_specs=[
            pl.BlockSpec((BN, l), lambda i, j: (i, 0)),
            pl.BlockSpec((BN, l), lambda i, j: (j, 0)),
        ],
        out_specs=pl.BlockSpec((8, 128), lambda i, j: (0, 0)),
        out_shape=jax.ShapeDtypeStruct((8, 128), jnp.float32),
        scratch_shapes=[pltpu.SMEM((1,), jnp.float32)],
    )(mu, mu)


def _edge_loss(zs, zd, src, dst):
    e, l = zs.shape
    BE = 4096
    nblk = e // BE

    def body(zsr, zdr, sr, dr, out, acc):
        i = pl.program_id(0)

        @pl.when(i == 0)
        def _():
            acc[0] = 0.0
            acc[1] = 0.0

        p = jnp.sum(zsr[...] * zdr[...], axis=1)
        m = (sr[...] != dr[...]).astype(jnp.float32)
        acc[0] = acc[0] + jnp.sum(m * _softplus(p))
        acc[1] = acc[1] + jnp.sum(m * p)

        @pl.when(i == nblk - 1)
        def _():
            out[...] = _scalar_tile([acc[0], acc[1]])

    return pl.pallas_call(
        body,
        grid=(nblk,),
        in_specs=[
            pl.BlockSpec((BE, l), lambda i: (i, 0)),
            pl.BlockSpec((BE, l), lambda i: (i, 0)),
            pl.BlockSpec((BE,), lambda i: (i,)),
            pl.BlockSpec((BE,), lambda i: (i,)),
        ],
        out_specs=pl.BlockSpec((8, 128), lambda i: (0, 0)),
        out_shape=jax.ShapeDtypeStruct((8, 128), jnp.float32),
        scratch_shapes=[pltpu.SMEM((2,), jnp.float32)],
    )(zs, zd, src, dst)


def _sc_mesh():
    return plsc.VectorSubcoreMesh(core_axis_name="c", subcore_axis_name="s")


def _zero_rows(rows, d):
    z = jnp.zeros((16,), jnp.float32)

    def zb(i, carry):
        for j in range(d // 16):
            rows[i, pl.ds(j * 16, 16)] = z
        return carry

    lax.fori_loop(jnp.int32(0), jnp.int32(_CH), zb, jnp.int32(0))


def _sc_degree(src2):
    rows_per_tile = _N // _NS
    nch = _E // _NW // _CH

    @functools.partial(
        pl.kernel,
        out_type=jax.ShapeDtypeStruct((_NC, _N, 128), jnp.float32),
        mesh=_sc_mesh(),
        scratch_types=[
            pltpu.VMEM((nch, _CH), jnp.int32),
            pltpu.VMEM((_CH, 128), jnp.float32),
            pltpu.VMEM_SHARED((_N, 128), jnp.float32),
            pltpu.SemaphoreType.DMA,
        ],
    )
    def k(src_hbm, out_hbm, sidx, ones_v, acc_sh, sem):
        cid = lax.axis_index("c")
        sid = lax.axis_index("s")
        crow0 = (cid * _NS + sid) * nch
        pltpu.sync_copy(src_hbm.at[pl.ds(crow0, nch)], sidx)
        _zero_rows(ones_v, 128)
        for r in range(rows_per_tile // _CH):
            pltpu.sync_copy(
                ones_v, acc_sh.at[pl.ds(sid * rows_per_tile + r * _CH, _CH)])
        one = jnp.ones((16,), jnp.float32)

        def ob(i, carry):
            for j in range(128 // 16):
                ones_v[i, pl.ds(j * 16, 16)] = one
            return carry

        lax.fori_loop(jnp.int32(0), jnp.int32(_CH), ob, jnp.int32(0))
        plsc.subcore_barrier()
        for kk in range(nch):
            pltpu.sync_copy(ones_v, acc_sh.at[sidx.at[kk]], add=True)
        plsc.subcore_barrier()
        pltpu.sync_copy(
            acc_sh.at[pl.ds(sid * rows_per_tile, rows_per_tile)],
            out_hbm.at[cid, pl.ds(sid * rows_per_tile, rows_per_tile)])

    return k(src2)


def _sc_gather_scatter(s, src2, dst2):
    n, d = s.shape
    rows_per_tile = n // _NS
    nch = _E // _NW // _CH

    @functools.partial(
        pl.kernel,
        out_type=jax.ShapeDtypeStruct((_NC, n, d), jnp.float32),
        mesh=_sc_mesh(),
        scratch_types=[
            pltpu.VMEM((nch, _CH), jnp.int32),
            pltpu.VMEM((nch, _CH), jnp.int32),
            pltpu.VMEM((_CH, d), jnp.float32),
            pltpu.VMEM((_CH, d), jnp.float32),
            pltpu.VMEM_SHARED((n, d), jnp.float32),
            pltpu.SemaphoreType.DMA,
            pltpu.SemaphoreType.DMA,
        ],
    )
    def k(s_hbm, src_hbm, dst_hbm, out_hbm, sidx, didx, rows0, rows1,
          acc_sh, sem0, sem1):
        cid = lax.axis_index("c")
        sid = lax.axis_index("s")
        crow0 = (cid * _NS + sid) * nch
        pltpu.sync_copy(src_hbm.at[pl.ds(crow0, nch)], sidx)
        pltpu.sync_copy(dst_hbm.at[pl.ds(crow0, nch)], didx)
        _zero_rows(rows0, d)
        for r in range(rows_per_tile // _CH):
            pltpu.sync_copy(
                rows0, acc_sh.at[pl.ds(sid * rows_per_tile + r * _CH, _CH)])
        plsc.subcore_barrier()
        bufs = (rows0, rows1)
        sems = (sem0, sem1)
        pend = pltpu.async_copy(s_hbm.at[didx.at[0]], rows0, sem0)
        for kk in range(nch):
            pend.wait()
            if kk + 1 < nch:
                pend = pltpu.async_copy(
                    s_hbm.at[didx.at[kk + 1]],
                    bufs[(kk + 1) % 2], sems[(kk + 1) % 2])
            pltpu.sync_copy(bufs[kk % 2], acc_sh.at[sidx.at[kk]], add=True)
        plsc.subcore_barrier()
        pltpu.sync_copy(
            acc_sh.at[pl.ds(sid * rows_per_tile, rows_per_tile)],
            out_hbm.at[cid, pl.ds(sid * rows_per_tile, rows_per_tile)])

    return k(s, src2, dst2)


def _sc_edge_gather(mu, src2, dst2):
    n, d = mu.shape
    epw = _E // _NW
    nch = epw // _CH

    @functools.partial(
        pl.kernel,
        out_type=(
            jax.ShapeDtypeStruct((_E, d), jnp.float32),
            jax.ShapeDtypeStruct((_E, d), jnp.float32),
        ),
        mesh=_sc_mesh(),
        scratch_types=[
            pltpu.VMEM((nch, _CH), jnp.int32),
            pltpu.VMEM((nch, _CH), jnp.int32),
            pltpu.VMEM((_CH, d), jnp.float32),
            pltpu.VMEM((_CH, d), jnp.float32),
            pltpu.SemaphoreType.DMA,
            pltpu.SemaphoreType.DMA,
        ],
    )
    def k(mu_hbm, src_hbm, dst_hbm, zs_hbm, zd_hbm, sidx, didx,
          rows0, rows1, sem0, sem1):
        cid = lax.axis_index("c")
        sid = lax.axis_index("s")
        crow0 = (cid * _NS + sid) * nch
        base0 = (cid * _NS + sid) * epw
        pltpu.sync_copy(src_hbm.at[pl.ds(crow0, nch)], sidx)
        pltpu.sync_copy(dst_hbm.at[pl.ds(crow0, nch)], didx)
        bufs = (rows0, rows1)
        sems = (sem0, sem1)
        jobs = []
        for kk in range(nch):
            jobs.append((sidx, kk, zs_hbm))
            jobs.append((didx, kk, zd_hbm))
        pend = pltpu.async_copy(mu_hbm.at[jobs[0][0].at[jobs[0][1]]],
                                rows0, sem0)
        for q, (idxref, kk, out_hbm) in enumerate(jobs):
            pend.wait()
            if q + 1 < len(jobs):
                nidx, nkk, _ = jobs[q + 1]
                pend = pltpu.async_copy(
                    mu_hbm.at[nidx.at[nkk]],
                    bufs[(q + 1) % 2], sems[(q + 1) % 2])
            pltpu.sync_copy(bufs[q % 2],
                            out_hbm.at[pl.ds(base0 + kk * _CH, _CH)])

    return k(mu, src2, dst2)


def kernel(x, edge_index, W1, b1, W2, b2, Wg1, Wg2, Wg3):
    with jax.enable_x64(False):
        h, gae_loss = _run(x, edge_index, W1, b1, W2, b2, Wg1, Wg2, Wg3)
    return h.astype(jnp.float64), gae_loss.astype(jnp.float64)


def _run(x, edge_index, W1, b1, W2, b2, Wg1, Wg2, Wg3):
    n = x.shape[0]
    e = edge_index.shape[1]
    src = edge_index[0].astype(jnp.int32)
    dst = edge_index[1].astype(jnp.int32)
    x = x.astype(jnp.float32)
    W1 = W1.astype(jnp.float32)
    b1 = b1.astype(jnp.float32)
    W2 = W2.astype(jnp.float32)
    b2 = b2.astype(jnp.float32)
    Wg1 = Wg1.astype(jnp.float32)
    Wg2 = Wg2.astype(jnp.float32)
    Wg3 = Wg3.astype(jnp.float32)

    src2 = src.reshape(-1, _CH)
    dst2 = dst.reshape(-1, _CH)

    h = _mlp(x, W1, b1, W2, b2)
    degp = _sc_degree(src2)

    t1p, dinv16 = _proj_scale(h, Wg1, degp)
    g1 = _sc_gather_scatter(t1p, src2, dst2)
    U = _hid_proj(g1, t1p, Wg2, Wg3, dinv16)
    g2 = _sc_gather_scatter(U, src2, dst2)
    mu, sc1 = _mu_kld(g2, U, dinv16)

    zs, zd = _sc_edge_gather(mu, src2, dst2)
    sc2 = _edge_loss(zs, zd, src, dst)
    sall = _dense_loss(mu)

    kld_sum = sc1[0, 0]
    diag_sp = sc1[0, 1]
    diag_p = sc1[0, 2]
    edge_sp = sc2[0, 0]
    edge_p = sc2[0, 1]
    all_sp = sall[0, 0]

    total = float(n) * float(n)
    s_edges = float(e)
    pos_weight = (total - s_edges) / s_edges
    norm = total / ((total - s_edges) * 2.0)

    bce = (all_sp
           + (pos_weight - 1.0) * (edge_sp + diag_sp)
           - pos_weight * (edge_p + diag_p)) / total
    kld = (-0.5) * kld_sum / total
    gae_loss = norm * bce + kld
    return h, gae_loss


# speedup vs baseline: 166.1716x; 1.0901x over previous
import functools

import jax
import jax.numpy as jnp
from jax import lax
from jax.experimental import pallas as pl
from jax.experimental.pallas import tpu as pltpu
from jax.experimental.pallas import tpu_sc as plsc

_N = 4096
_D = 128
_L = 64
_E = 65536

_NC = 2
_NS = 16
_NW = _NC * _NS
_CH = 128


def _softplus(x):
    return jnp.maximum(x, 0.0) + jnp.log1p(jnp.exp(-jnp.abs(x)))


def _scalar_tile(vals):
    r = lax.broadcasted_iota(jnp.int32, (8, 128), 0)
    c = lax.broadcasted_iota(jnp.int32, (8, 128), 1)
    out = jnp.zeros((8, 128), jnp.float32)
    for i, v in enumerate(vals):
        out = out + jnp.where((r == 0) & (c == i), v, 0.0)
    return out


def _dinv_block(degp_blk):
    deg = 1.0 + degp_blk[0][:, 0:1] + degp_blk[1][:, 0:1]
    return lax.rsqrt(deg)


def _mlp(x, W1, b1, W2, b2):
    n, d = x.shape
    BN = 512

    def body(xr, w1r, b1r, w2r, b2r, out):
        hb = jnp.maximum(
            jnp.dot(xr[...], w1r[...], preferred_element_type=jnp.float32, precision=lax.Precision.HIGHEST)
            + b1r[...], 0.0)
        out[...] = (
            jnp.dot(hb, w2r[...], preferred_element_type=jnp.float32, precision=lax.Precision.HIGHEST)
            + b2r[...])

    return pl.pallas_call(
        body,
        grid=(n // BN,),
        in_specs=[
            pl.BlockSpec((BN, d), lambda i: (i, 0)),
            pl.BlockSpec(W1.shape, lambda i: (0, 0)),
            pl.BlockSpec((1, b1.shape[0]), lambda i: (0, 0)),
            pl.BlockSpec(W2.shape, lambda i: (0, 0)),
            pl.BlockSpec((1, b2.shape[0]), lambda i: (0, 0)),
        ],
        out_specs=pl.BlockSpec((BN, W2.shape[1]), lambda i: (i, 0)),
        out_shape=jax.ShapeDtypeStruct((n, W2.shape[1]), jnp.float32),
    )(x, W1, b1.reshape(1, -1), W2, b2.reshape(1, -1))


def _proj_scale(h, Wg1, degp):
    n, l = h.shape
    BN = 512

    def body(hr, wr, dgr, out, dv_out):
        dinv = _dinv_block(dgr[...])
        t = jnp.dot(hr[...], wr[...], preferred_element_type=jnp.float32, precision=lax.Precision.HIGHEST) * dinv
        out[...] = jnp.concatenate(
            [t, jnp.zeros((BN, 128 - l), jnp.float32)], axis=1)
        dv_out[...] = jnp.broadcast_to(dinv, (BN, 16))

    return pl.pallas_call(
        body,
        grid=(n // BN,),
        in_specs=[
            pl.BlockSpec((BN, l), lambda i: (i, 0)),
            pl.BlockSpec(Wg1.shape, lambda i: (0, 0)),
            pl.BlockSpec((2, BN, 128), lambda i: (0, i, 0)),
        ],
        out_specs=[
            pl.BlockSpec((BN, 128), lambda i: (i, 0)),
            pl.BlockSpec((BN, 16), lambda i: (i, 0)),
        ],
        out_shape=[
            jax.ShapeDtypeStruct((n, 128), jnp.float32),
            jax.ShapeDtypeStruct((n, 16), jnp.float32),
        ],
    )(h, Wg1, degp)


def _hid_proj(g1, t1p, Wg2, Wg3, dinv16):
    n, _ = t1p.shape
    l = Wg2.shape[0]
    BN = 512

    def body(g1r, t1r, dvr, w2r, w3r, out):
        dinv = dvr[...][:, 0:1]
        gs = g1r[0] + g1r[1] + t1r[...]
        hid = jnp.maximum(dinv * gs[:, :l], 0.0)
        u2 = jnp.dot(hid, w2r[...], preferred_element_type=jnp.float32, precision=lax.Precision.HIGHEST) * dinv
        u3 = jnp.dot(hid, w3r[...], preferred_element_type=jnp.float32, precision=lax.Precision.HIGHEST) * dinv
        out[...] = jnp.concatenate([u2, u3], axis=1)

    return pl.pallas_call(
        body,
        grid=(n // BN,),
        in_specs=[
            pl.BlockSpec((2, BN, 128), lambda i: (0, i, 0)),
            pl.BlockSpec((BN, 128), lambda i: (i, 0)),
            pl.BlockSpec((BN, 16), lambda i: (i, 0)),
            pl.BlockSpec(Wg2.shape, lambda i: (0, 0)),
            pl.BlockSpec(Wg3.shape, lambda i: (0, 0)),
        ],
        out_specs=pl.BlockSpec((BN, 2 * l), lambda i: (i, 0)),
        out_shape=jax.ShapeDtypeStruct((n, 2 * l), jnp.float32),
    )(g1, t1p, dinv16, Wg2, Wg3)


def _mu_kld(g2, U, dinv16):
    n, d2 = U.shape
    l = d2 // 2
    BN = 512
    nblk = n // BN

    def body(g2r, ur, dvr, mu_out, sc_out, acc):
        i = pl.program_id(0)

        @pl.when(i == 0)
        def _():
            acc[0] = 0.0
            acc[1] = 0.0
            acc[2] = 0.0

        dinv = dvr[...][:, 0:1]
        v = dinv * (g2r[0] + g2r[1] + ur[...])
        mu = v[:, :l]
        lv = v[:, l:]
        mu_out[...] = jnp.concatenate(
            [mu, jnp.zeros((BN, d2 - l), jnp.float32)], axis=1)
        kt = jnp.sum(1.0 + 2.0 * lv - mu * mu - jnp.exp(2.0 * lv))
        q = jnp.sum(mu * mu, axis=1)
        acc[0] = acc[0] + kt
        acc[1] = acc[1] + jnp.sum(_softplus(q))
        acc[2] = acc[2] + jnp.sum(q)

        @pl.when(i == nblk - 1)
        def _():
            sc_out[...] = _scalar_tile([acc[0], acc[1], acc[2]])

    return pl.pallas_call(
        body,
        grid=(nblk,),
        in_specs=[
            pl.BlockSpec((2, BN, d2), lambda i: (0, i, 0)),
            pl.BlockSpec((BN, d2), lambda i: (i, 0)),
            pl.BlockSpec((BN, 16), lambda i: (i, 0)),
        ],
        out_specs=[
            pl.BlockSpec((BN, d2), lambda i: (i, 0)),
            pl.BlockSpec((8, 128), lambda i: (0, 0)),
        ],
        out_shape=[
            jax.ShapeDtypeStruct((n, d2), jnp.float32),
            jax.ShapeDtypeStruct((8, 128), jnp.float32),
        ],
        scratch_shapes=[pltpu.SMEM((4,), jnp.float32)],
    )(g2, U, dinv16)


def _dense_loss(mu):
    n, l = mu.shape
    BN = 512
    nblk = n // BN

    def body(ar, br, out, acc):
        i = pl.program_id(0)
        j = pl.program_id(1)

        @pl.when((i == 0) & (j == 0))
        def _():
            acc[0] = 0.0

        p = lax.dot_general(
            ar[...][:, :_L], br[...][:, :_L], (((1,), (1,)), ((), ())),
            preferred_element_type=jnp.float32,
            precision=lax.Precision.HIGHEST)
        acc[0] = acc[0] + jnp.sum(_softplus(p))

        @pl.when((i == nblk - 1) & (j == nblk - 1))
        def _():
            out[...] = _scalar_tile([acc[0]])

    return pl.pallas_call(
        body,
        grid=(nblk, nblk),
        in_specs=[
            pl.BlockSpec((BN, l), lambda i, j: (i, 0)),
            pl.BlockSpec((BN, l), lambda i, j: (j, 0)),
        ],
        out_specs=pl.BlockSpec((8, 128), lambda i, j: (0, 0)),
        out_shape=jax.ShapeDtypeStruct((8, 128), jnp.float32),
        scratch_shapes=[pltpu.SMEM((1,), jnp.float32)],
    )(mu, mu)


def _edge_loss(zs, zd, src, dst):
    e, l = zs.shape
    BE = 4096
    nblk = e // BE

    def body(zsr, zdr, sr, dr, out, acc):
        i = pl.program_id(0)

        @pl.when(i == 0)
        def _():
            acc[0] = 0.0
            acc[1] = 0.0

        p = jnp.sum(zsr[...] * zdr[...], axis=1)
        m = (sr[...] != dr[...]).astype(jnp.float32)
        acc[0] = acc[0] + jnp.sum(m * _softplus(p))
        acc[1] = acc[1] + jnp.sum(m * p)

        @pl.when(i == nblk - 1)
        def _():
            out[...] = _scalar_tile([acc[0], acc[1]])

    return pl.pallas_call(
        body,
        grid=(nblk,),
        in_specs=[
            pl.BlockSpec((BE, l), lambda i: (i, 0)),
            pl.BlockSpec((BE, l), lambda i: (i, 0)),
            pl.BlockSpec((BE,), lambda i: (i,)),
            pl.BlockSpec((BE,), lambda i: (i,)),
        ],
        out_specs=pl.BlockSpec((8, 128), lambda i: (0, 0)),
        out_shape=jax.ShapeDtypeStruct((8, 128), jnp.float32),
        scratch_shapes=[pltpu.SMEM((2,), jnp.float32)],
    )(zs, zd, src, dst)


def _sc_mesh():
    return plsc.VectorSubcoreMesh(core_axis_name="c", subcore_axis_name="s")


def _zero_rows(rows, d):
    z = jnp.zeros((16,), jnp.float32)

    def zb(i, carry):
        for j in range(d // 16):
            rows[i, pl.ds(j * 16, 16)] = z
        return carry

    lax.fori_loop(jnp.int32(0), jnp.int32(_CH), zb, jnp.int32(0))


def _sc_degree(src2):
    rows_per_tile = _N // _NS
    nch = _E // _NW // _CH

    @functools.partial(
        pl.kernel,
        out_type=jax.ShapeDtypeStruct((_NC, _N, 128), jnp.float32),
        mesh=_sc_mesh(),
        scratch_types=[
            pltpu.VMEM((nch, _CH), jnp.int32),
            pltpu.VMEM((_CH, 128), jnp.float32),
            pltpu.VMEM_SHARED((_N, 128), jnp.float32),
            pltpu.SemaphoreType.DMA,
        ],
    )
    def k(src_hbm, out_hbm, sidx, ones_v, acc_sh, sem):
        cid = lax.axis_index("c")
        sid = lax.axis_index("s")
        crow0 = (cid * _NS + sid) * nch
        pltpu.sync_copy(src_hbm.at[pl.ds(crow0, nch)], sidx)
        _zero_rows(ones_v, 128)
        for r in range(rows_per_tile // _CH):
            pltpu.sync_copy(
                ones_v, acc_sh.at[pl.ds(sid * rows_per_tile + r * _CH, _CH)])
        one = jnp.ones((16,), jnp.float32)

        def ob(i, carry):
            for j in range(128 // 16):
                ones_v[i, pl.ds(j * 16, 16)] = one
            return carry

        lax.fori_loop(jnp.int32(0), jnp.int32(_CH), ob, jnp.int32(0))
        plsc.subcore_barrier()
        for kk in range(nch):
            pltpu.sync_copy(ones_v, acc_sh.at[sidx.at[kk]], add=True)
        plsc.subcore_barrier()
        pltpu.sync_copy(
            acc_sh.at[pl.ds(sid * rows_per_tile, rows_per_tile)],
            out_hbm.at[cid, pl.ds(sid * rows_per_tile, rows_per_tile)])

    return k(src2)


def _sc_gather_scatter(s, src2, dst2):
    n, d = s.shape
    rows_per_tile = n // _NS
    nch = _E // _NW // _CH

    @functools.partial(
        pl.kernel,
        out_type=jax.ShapeDtypeStruct((_NC, n, d), jnp.float32),
        mesh=_sc_mesh(),
        scratch_types=[
            pltpu.VMEM((nch, _CH), jnp.int32),
            pltpu.VMEM((nch, _CH), jnp.int32),
            pltpu.VMEM((_CH, d), jnp.float32),
            pltpu.VMEM((_CH, d), jnp.float32),
            pltpu.VMEM_SHARED((n, d), jnp.float32),
            pltpu.SemaphoreType.DMA,
            pltpu.SemaphoreType.DMA,
        ],
    )
    def k(s_hbm, src_hbm, dst_hbm, out_hbm, sidx, didx, rows0, rows1,
          acc_sh, sem0, sem1):
        cid = lax.axis_index("c")
        sid = lax.axis_index("s")
        crow0 = (cid * _NS + sid) * nch
        pltpu.sync_copy(src_hbm.at[pl.ds(crow0, nch)], sidx)
        pltpu.sync_copy(dst_hbm.at[pl.ds(crow0, nch)], didx)
        _zero_rows(rows0, d)
        for r in range(rows_per_tile // _CH):
            pltpu.sync_copy(
                rows0, acc_sh.at[pl.ds(sid * rows_per_tile + r * _CH, _CH)])
        plsc.subcore_barrier()
        bufs = (rows0, rows1)
        sems = (sem0, sem1)
        pend = pltpu.async_copy(s_hbm.at[didx.at[0]], rows0, sem0)
        for kk in range(nch):
            pend.wait()
            if kk + 1 < nch:
                pend = pltpu.async_copy(
                    s_hbm.at[didx.at[kk + 1]],
                    bufs[(kk + 1) % 2], sems[(kk + 1) % 2])
            pltpu.sync_copy(bufs[kk % 2], acc_sh.at[sidx.at[kk]], add=True)
        plsc.subcore_barrier()
        pltpu.sync_copy(
            acc_sh.at[pl.ds(sid * rows_per_tile, rows_per_tile)],
            out_hbm.at[cid, pl.ds(sid * rows_per_tile, rows_per_tile)])

    return k(s, src2, dst2)


def _sc_edge_gather(mu, src2, dst2):
    n, d = mu.shape
    epw = _E // _NW
    nch = epw // _CH

    @functools.partial(
        pl.kernel,
        out_type=(
            jax.ShapeDtypeStruct((_E, d), jnp.float32),
            jax.ShapeDtypeStruct((_E, d), jnp.float32),
        ),
        mesh=_sc_mesh(),
        scratch_types=[
            pltpu.VMEM((nch, _CH), jnp.int32),
            pltpu.VMEM((nch, _CH), jnp.int32),
            pltpu.VMEM((_CH, d), jnp.float32),
            pltpu.VMEM((_CH, d), jnp.float32),
            pltpu.VMEM((_CH, d), jnp.float32),
            pltpu.VMEM((_CH, d), jnp.float32),
            pltpu.SemaphoreType.DMA,
            pltpu.SemaphoreType.DMA,
            pltpu.SemaphoreType.DMA,
            pltpu.SemaphoreType.DMA,
            pltpu.SemaphoreType.DMA,
            pltpu.SemaphoreType.DMA,
            pltpu.SemaphoreType.DMA,
            pltpu.SemaphoreType.DMA,
        ],
    )
    def k(mu_hbm, src_hbm, dst_hbm, zs_hbm, zd_hbm, sidx, didx,
          rows0, rows1, rows2, rows3,
          gsem0, gsem1, gsem2, gsem3, ssem0, ssem1, ssem2, ssem3):
        cid = lax.axis_index("c")
        sid = lax.axis_index("s")
        crow0 = (cid * _NS + sid) * nch
        base0 = (cid * _NS + sid) * epw
        pltpu.sync_copy(src_hbm.at[pl.ds(crow0, nch)], sidx)
        pltpu.sync_copy(dst_hbm.at[pl.ds(crow0, nch)], didx)
        bufs = (rows0, rows1, rows2, rows3)
        gsems = (gsem0, gsem1, gsem2, gsem3)
        ssems = (ssem0, ssem1, ssem2, ssem3)
        jobs = []
        for kk in range(nch):
            jobs.append((sidx, kk, zs_hbm))
            jobs.append((didx, kk, zd_hbm))
        nj = len(jobs)
        lag = 2
        gd = [None] * nj
        sd = [None] * nj
        for q in range(nj + lag):
            if q < nj:
                b = q % 4
                if q >= 4:
                    sd[q - 4].wait()
                idxref, kk, _ = jobs[q]
                gd[q] = pltpu.async_copy(
                    mu_hbm.at[idxref.at[kk]], bufs[b], gsems[b])
            if q >= lag:
                p = q - lag
                b = p % 4
                _, kk, out_hbm = jobs[p]
                gd[p].wait()
                sd[p] = pltpu.async_copy(
                    bufs[b], out_hbm.at[pl.ds(base0 + kk * _CH, _CH)],
                    ssems[b])
        for p in range(nj - 4, nj):
            sd[p].wait()

    return k(mu, src2, dst2)


def kernel(x, edge_index, W1, b1, W2, b2, Wg1, Wg2, Wg3):
    with jax.enable_x64(False):
        h, gae_loss = _run(x, edge_index, W1, b1, W2, b2, Wg1, Wg2, Wg3)
    return h.astype(jnp.float64), gae_loss.astype(jnp.float64)


def _run(x, edge_index, W1, b1, W2, b2, Wg1, Wg2, Wg3):
    n = x.shape[0]
    e = edge_index.shape[1]
    src = edge_index[0].astype(jnp.int32)
    dst = edge_index[1].astype(jnp.int32)
    x = x.astype(jnp.float32)
    W1 = W1.astype(jnp.float32)
    b1 = b1.astype(jnp.float32)
    W2 = W2.astype(jnp.float32)
    b2 = b2.astype(jnp.float32)
    Wg1 = Wg1.astype(jnp.float32)
    Wg2 = Wg2.astype(jnp.float32)
    Wg3 = Wg3.astype(jnp.float32)

    src2 = src.reshape(-1, _CH)
    dst2 = dst.reshape(-1, _CH)

    h = _mlp(x, W1, b1, W2, b2)
    degp = _sc_degree(src2)

    t1p, dinv16 = _proj_scale(h, Wg1, degp)
    g1 = _sc_gather_scatter(t1p, src2, dst2)
    U = _hid_proj(g1, t1p, Wg2, Wg3, dinv16)
    g2 = _sc_gather_scatter(U, src2, dst2)
    mu, sc1 = _mu_kld(g2, U, dinv16)

    sall = _dense_loss(mu)
    zs, zd = _sc_edge_gather(mu, src2, dst2)
    sc2 = _edge_loss(zs, zd, src, dst)

    kld_sum = sc1[0, 0]
    diag_sp = sc1[0, 1]
    diag_p = sc1[0, 2]
    edge_sp = sc2[0, 0]
    edge_p = sc2[0, 1]
    all_sp = sall[0, 0]

    total = float(n) * float(n)
    s_edges = float(e)
    pos_weight = (total - s_edges) / s_edges
    norm = total / ((total - s_edges) * 2.0)

    bce = (all_sp
           + (pos_weight - 1.0) * (edge_sp + diag_sp)
           - pos_weight * (edge_p + diag_p)) / total
    kld = (-0.5) * kld_sum / total
    gae_loss = norm * bce + kld
    return h, gae_loss


# speedup vs baseline: 174.2725x; 1.0488x over previous
import functools

import jax
import jax.numpy as jnp
from jax import lax
from jax.experimental import pallas as pl
from jax.experimental.pallas import tpu as pltpu
from jax.experimental.pallas import tpu_sc as plsc

_N = 4096
_D = 128
_L = 64
_E = 65536

_NC = 2
_NS = 16
_NW = _NC * _NS
_CH = 128


def _softplus(x):
    return jnp.maximum(x, 0.0) + jnp.log1p(jnp.exp(-jnp.abs(x)))


def _scalar_tile(vals):
    r = lax.broadcasted_iota(jnp.int32, (8, 128), 0)
    c = lax.broadcasted_iota(jnp.int32, (8, 128), 1)
    out = jnp.zeros((8, 128), jnp.float32)
    for i, v in enumerate(vals):
        out = out + jnp.where((r == 0) & (c == i), v, 0.0)
    return out


def _dinv_block(degp_blk):
    deg = 1.0 + degp_blk[0][:, 0:1] + degp_blk[1][:, 0:1]
    return lax.rsqrt(deg)


def _mlp(x, W1, b1, W2, b2):
    n, d = x.shape
    BN = 512

    def body(xr, w1r, b1r, w2r, b2r, out):
        hb = jnp.maximum(
            jnp.dot(xr[...], w1r[...], preferred_element_type=jnp.float32, precision=lax.Precision.HIGHEST)
            + b1r[...], 0.0)
        out[...] = (
            jnp.dot(hb, w2r[...], preferred_element_type=jnp.float32, precision=lax.Precision.HIGHEST)
            + b2r[...])

    return pl.pallas_call(
        body,
        grid=(n // BN,),
        in_specs=[
            pl.BlockSpec((BN, d), lambda i: (i, 0)),
            pl.BlockSpec(W1.shape, lambda i: (0, 0)),
            pl.BlockSpec((1, b1.shape[0]), lambda i: (0, 0)),
            pl.BlockSpec(W2.shape, lambda i: (0, 0)),
            pl.BlockSpec((1, b2.shape[0]), lambda i: (0, 0)),
        ],
        out_specs=pl.BlockSpec((BN, W2.shape[1]), lambda i: (i, 0)),
        out_shape=jax.ShapeDtypeStruct((n, W2.shape[1]), jnp.float32),
    )(x, W1, b1.reshape(1, -1), W2, b2.reshape(1, -1))


def _proj_scale(h, Wg1, degp):
    n, l = h.shape
    BN = 512

    def body(hr, wr, dgr, out, dv_out):
        dinv = _dinv_block(dgr[...])
        t = jnp.dot(hr[...], wr[...], preferred_element_type=jnp.float32, precision=lax.Precision.HIGHEST) * dinv
        out[...] = jnp.concatenate(
            [t, jnp.zeros((BN, 128 - l), jnp.float32)], axis=1)
        dv_out[...] = jnp.broadcast_to(dinv, (BN, 16))

    return pl.pallas_call(
        body,
        grid=(n // BN,),
        in_specs=[
            pl.BlockSpec((BN, l), lambda i: (i, 0)),
            pl.BlockSpec(Wg1.shape, lambda i: (0, 0)),
            pl.BlockSpec((2, BN, 128), lambda i: (0, i, 0)),
        ],
        out_specs=[
            pl.BlockSpec((BN, 128), lambda i: (i, 0)),
            pl.BlockSpec((BN, 16), lambda i: (i, 0)),
        ],
        out_shape=[
            jax.ShapeDtypeStruct((n, 128), jnp.float32),
            jax.ShapeDtypeStruct((n, 16), jnp.float32),
        ],
    )(h, Wg1, degp)


def _hid_proj(g1, t1p, Wg2, Wg3, dinv16):
    n, _ = t1p.shape
    l = Wg2.shape[0]
    BN = 512

    def body(g1r, t1r, dvr, w2r, w3r, out):
        dinv = dvr[...][:, 0:1]
        gs = g1r[0] + g1r[1] + t1r[...]
        hid = jnp.maximum(dinv * gs[:, :l], 0.0)
        u2 = jnp.dot(hid, w2r[...], preferred_element_type=jnp.float32, precision=lax.Precision.HIGHEST) * dinv
        u3 = jnp.dot(hid, w3r[...], preferred_element_type=jnp.float32, precision=lax.Precision.HIGHEST) * dinv
        out[...] = jnp.concatenate([u2, u3], axis=1)

    return pl.pallas_call(
        body,
        grid=(n // BN,),
        in_specs=[
            pl.BlockSpec((2, BN, 128), lambda i: (0, i, 0)),
            pl.BlockSpec((BN, 128), lambda i: (i, 0)),
            pl.BlockSpec((BN, 16), lambda i: (i, 0)),
            pl.BlockSpec(Wg2.shape, lambda i: (0, 0)),
            pl.BlockSpec(Wg3.shape, lambda i: (0, 0)),
        ],
        out_specs=pl.BlockSpec((BN, 2 * l), lambda i: (i, 0)),
        out_shape=jax.ShapeDtypeStruct((n, 2 * l), jnp.float32),
    )(g1, t1p, dinv16, Wg2, Wg3)


def _mu_kld(g2, U, dinv16):
    n, d2 = U.shape
    l = d2 // 2
    BN = 512
    nblk = n // BN

    def body(g2r, ur, dvr, mu_out, sc_out, acc):
        i = pl.program_id(0)

        @pl.when(i == 0)
        def _():
            acc[0] = 0.0
            acc[1] = 0.0
            acc[2] = 0.0

        dinv = dvr[...][:, 0:1]
        v = dinv * (g2r[0] + g2r[1] + ur[...])
        mu = v[:, :l]
        lv = v[:, l:]
        mu_out[...] = jnp.concatenate(
            [mu, jnp.zeros((BN, d2 - l), jnp.float32)], axis=1)
        kt = jnp.sum(1.0 + 2.0 * lv - mu * mu - jnp.exp(2.0 * lv))
        q = jnp.sum(mu * mu, axis=1)
        acc[0] = acc[0] + kt
        acc[1] = acc[1] + jnp.sum(_softplus(q))
        acc[2] = acc[2] + jnp.sum(q)

        @pl.when(i == nblk - 1)
        def _():
            sc_out[...] = _scalar_tile([acc[0], acc[1], acc[2]])

    return pl.pallas_call(
        body,
        grid=(nblk,),
        in_specs=[
            pl.BlockSpec((2, BN, d2), lambda i: (0, i, 0)),
            pl.BlockSpec((BN, d2), lambda i: (i, 0)),
            pl.BlockSpec((BN, 16), lambda i: (i, 0)),
        ],
        out_specs=[
            pl.BlockSpec((BN, d2), lambda i: (i, 0)),
            pl.BlockSpec((8, 128), lambda i: (0, 0)),
        ],
        out_shape=[
            jax.ShapeDtypeStruct((n, d2), jnp.float32),
            jax.ShapeDtypeStruct((8, 128), jnp.float32),
        ],
        scratch_shapes=[pltpu.SMEM((4,), jnp.float32)],
    )(g2, U, dinv16)


def _dense_loss(mu):
    n, l = mu.shape
    BN = 512
    nblk = n // BN

    def body(ar, br, out, acc):
        i = pl.program_id(0)
        j = pl.program_id(1)

        @pl.when((i == 0) & (j == 0))
        def _():
            acc[0] = 0.0

        @pl.when(j >= i)
        def _():
            p = lax.dot_general(
                ar[...][:, :_L], br[...][:, :_L], (((1,), (1,)), ((), ())),
                preferred_element_type=jnp.float32,
                precision=lax.Precision.HIGHEST)
            w = jnp.where(j == i, 1.0, 2.0)
            acc[0] = acc[0] + w * jnp.sum(_softplus(p))

        @pl.when((i == nblk - 1) & (j == nblk - 1))
        def _():
            out[...] = _scalar_tile([acc[0]])

    return pl.pallas_call(
        body,
        grid=(nblk, nblk),
        in_specs=[
            pl.BlockSpec((BN, l), lambda i, j: (i, 0)),
            pl.BlockSpec((BN, l), lambda i, j: (j, 0)),
        ],
        out_specs=pl.BlockSpec((8, 128), lambda i, j: (0, 0)),
        out_shape=jax.ShapeDtypeStruct((8, 128), jnp.float32),
        scratch_shapes=[pltpu.SMEM((1,), jnp.float32)],
    )(mu, mu)


def _edge_loss(zs, zd, src, dst):
    e, l = zs.shape
    BE = 4096
    nblk = e // BE

    def body(zsr, zdr, sr, dr, out, acc):
        i = pl.program_id(0)

        @pl.when(i == 0)
        def _():
            acc[0] = 0.0
            acc[1] = 0.0

        p = jnp.sum(zsr[...] * zdr[...], axis=1)
        m = (sr[...] != dr[...]).astype(jnp.float32)
        acc[0] = acc[0] + jnp.sum(m * _softplus(p))
        acc[1] = acc[1] + jnp.sum(m * p)

        @pl.when(i == nblk - 1)
        def _():
            out[...] = _scalar_tile([acc[0], acc[1]])

    return pl.pallas_call(
        body,
        grid=(nblk,),
        in_specs=[
            pl.BlockSpec((BE, l), lambda i: (i, 0)),
            pl.BlockSpec((BE, l), lambda i: (i, 0)),
            pl.BlockSpec((BE,), lambda i: (i,)),
            pl.BlockSpec((BE,), lambda i: (i,)),
        ],
        out_specs=pl.BlockSpec((8, 128), lambda i: (0, 0)),
        out_shape=jax.ShapeDtypeStruct((8, 128), jnp.float32),
        scratch_shapes=[pltpu.SMEM((2,), jnp.float32)],
    )(zs, zd, src, dst)


def _sc_mesh():
    return plsc.VectorSubcoreMesh(core_axis_name="c", subcore_axis_name="s")


def _zero_rows(rows, d):
    z = jnp.zeros((16,), jnp.float32)

    def zb(i, carry):
        for j in range(d // 16):
            rows[i, pl.ds(j * 16, 16)] = z
        return carry

    lax.fori_loop(jnp.int32(0), jnp.int32(_CH), zb, jnp.int32(0))


def _sc_degree(src2):
    rows_per_tile = _N // _NS
    nch = _E // _NW // _CH

    @functools.partial(
        pl.kernel,
        out_type=jax.ShapeDtypeStruct((_NC, _N, 128), jnp.float32),
        mesh=_sc_mesh(),
        scratch_types=[
            pltpu.VMEM((nch, _CH), jnp.int32),
            pltpu.VMEM((_CH, 128), jnp.float32),
            pltpu.VMEM_SHARED((_N, 128), jnp.float32),
            pltpu.SemaphoreType.DMA,
        ],
    )
    def k(src_hbm, out_hbm, sidx, ones_v, acc_sh, sem):
        cid = lax.axis_index("c")
        sid = lax.axis_index("s")
        crow0 = (cid * _NS + sid) * nch
        pltpu.sync_copy(src_hbm.at[pl.ds(crow0, nch)], sidx)
        _zero_rows(ones_v, 128)
        for r in range(rows_per_tile // _CH):
            pltpu.sync_copy(
                ones_v, acc_sh.at[pl.ds(sid * rows_per_tile + r * _CH, _CH)])
        one = jnp.ones((16,), jnp.float32)

        def ob(i, carry):
            for j in range(128 // 16):
                ones_v[i, pl.ds(j * 16, 16)] = one
            return carry

        lax.fori_loop(jnp.int32(0), jnp.int32(_CH), ob, jnp.int32(0))
        plsc.subcore_barrier()
        for kk in range(nch):
            pltpu.sync_copy(ones_v, acc_sh.at[sidx.at[kk]], add=True)
        plsc.subcore_barrier()
        pltpu.sync_copy(
            acc_sh.at[pl.ds(sid * rows_per_tile, rows_per_tile)],
            out_hbm.at[cid, pl.ds(sid * rows_per_tile, rows_per_tile)])

    return k(src2)


def _sc_gather_scatter(s, src2, dst2):
    n, d = s.shape
    rows_per_tile = n // _NS
    nch = _E // _NW // _CH

    @functools.partial(
        pl.kernel,
        out_type=jax.ShapeDtypeStruct((_NC, n, d), jnp.float32),
        mesh=_sc_mesh(),
        scratch_types=[
            pltpu.VMEM((nch, _CH), jnp.int32),
            pltpu.VMEM((nch, _CH), jnp.int32),
            pltpu.VMEM((_CH, d), jnp.float32),
            pltpu.VMEM((_CH, d), jnp.float32),
            pltpu.VMEM_SHARED((n, d), jnp.float32),
            pltpu.SemaphoreType.DMA,
            pltpu.SemaphoreType.DMA,
        ],
    )
    def k(s_hbm, src_hbm, dst_hbm, out_hbm, sidx, didx, rows0, rows1,
          acc_sh, sem0, sem1):
        cid = lax.axis_index("c")
        sid = lax.axis_index("s")
        crow0 = (cid * _NS + sid) * nch
        pltpu.sync_copy(src_hbm.at[pl.ds(crow0, nch)], sidx)
        pltpu.sync_copy(dst_hbm.at[pl.ds(crow0, nch)], didx)
        _zero_rows(rows0, d)
        for r in range(rows_per_tile // _CH):
            pltpu.sync_copy(
                rows0, acc_sh.at[pl.ds(sid * rows_per_tile + r * _CH, _CH)])
        plsc.subcore_barrier()
        bufs = (rows0, rows1)
        sems = (sem0, sem1)
        pend = pltpu.async_copy(s_hbm.at[didx.at[0]], rows0, sem0)
        for kk in range(nch):
            pend.wait()
            if kk + 1 < nch:
                pend = pltpu.async_copy(
                    s_hbm.at[didx.at[kk + 1]],
                    bufs[(kk + 1) % 2], sems[(kk + 1) % 2])
            pltpu.sync_copy(bufs[kk % 2], acc_sh.at[sidx.at[kk]], add=True)
        plsc.subcore_barrier()
        pltpu.sync_copy(
            acc_sh.at[pl.ds(sid * rows_per_tile, rows_per_tile)],
            out_hbm.at[cid, pl.ds(sid * rows_per_tile, rows_per_tile)])

    return k(s, src2, dst2)


def _sc_edge_gather(mu, src2, dst2):
    n, d = mu.shape
    epw = _E // _NW
    nch = epw // _CH

    @functools.partial(
        pl.kernel,
        out_type=(
            jax.ShapeDtypeStruct((_E, d), jnp.float32),
            jax.ShapeDtypeStruct((_E, d), jnp.float32),
        ),
        mesh=_sc_mesh(),
        scratch_types=[
            pltpu.VMEM((nch, _CH), jnp.int32),
            pltpu.VMEM((nch, _CH), jnp.int32),
            pltpu.VMEM((_CH, d), jnp.float32),
            pltpu.VMEM((_CH, d), jnp.float32),
            pltpu.VMEM((_CH, d), jnp.float32),
            pltpu.VMEM((_CH, d), jnp.float32),
            pltpu.SemaphoreType.DMA,
            pltpu.SemaphoreType.DMA,
            pltpu.SemaphoreType.DMA,
            pltpu.SemaphoreType.DMA,
            pltpu.SemaphoreType.DMA,
            pltpu.SemaphoreType.DMA,
            pltpu.SemaphoreType.DMA,
            pltpu.SemaphoreType.DMA,
        ],
    )
    def k(mu_hbm, src_hbm, dst_hbm, zs_hbm, zd_hbm, sidx, didx,
          rows0, rows1, rows2, rows3,
          gsem0, gsem1, gsem2, gsem3, ssem0, ssem1, ssem2, ssem3):
        cid = lax.axis_index("c")
        sid = lax.axis_index("s")
        crow0 = (cid * _NS + sid) * nch
        base0 = (cid * _NS + sid) * epw
        pltpu.sync_copy(src_hbm.at[pl.ds(crow0, nch)], sidx)
        pltpu.sync_copy(dst_hbm.at[pl.ds(crow0, nch)], didx)
        bufs = (rows0, rows1, rows2, rows3)
        gsems = (gsem0, gsem1, gsem2, gsem3)
        ssems = (ssem0, ssem1, ssem2, ssem3)
        jobs = []
        for kk in range(nch):
            jobs.append((sidx, kk, zs_hbm))
            jobs.append((didx, kk, zd_hbm))
        nj = len(jobs)
        lag = 2
        gd = [None] * nj
        sd = [None] * nj
        for q in range(nj + lag):
            if q < nj:
                b = q % 4
                if q >= 4:
                    sd[q - 4].wait()
                idxref, kk, _ = jobs[q]
                gd[q] = pltpu.async_copy(
                    mu_hbm.at[idxref.at[kk]], bufs[b], gsems[b])
            if q >= lag:
                p = q - lag
                b = p % 4
                _, kk, out_hbm = jobs[p]
                gd[p].wait()
                sd[p] = pltpu.async_copy(
                    bufs[b], out_hbm.at[pl.ds(base0 + kk * _CH, _CH)],
                    ssems[b])
        for p in range(nj - 4, nj):
            sd[p].wait()

    return k(mu, src2, dst2)


def kernel(x, edge_index, W1, b1, W2, b2, Wg1, Wg2, Wg3):
    with jax.enable_x64(False):
        h, gae_loss = _run(x, edge_index, W1, b1, W2, b2, Wg1, Wg2, Wg3)
    return h.astype(jnp.float64), gae_loss.astype(jnp.float64)


def _run(x, edge_index, W1, b1, W2, b2, Wg1, Wg2, Wg3):
    n = x.shape[0]
    e = edge_index.shape[1]
    src = edge_index[0].astype(jnp.int32)
    dst = edge_index[1].astype(jnp.int32)
    x = x.astype(jnp.float32)
    W1 = W1.astype(jnp.float32)
    b1 = b1.astype(jnp.float32)
    W2 = W2.astype(jnp.float32)
    b2 = b2.astype(jnp.float32)
    Wg1 = Wg1.astype(jnp.float32)
    Wg2 = Wg2.astype(jnp.float32)
    Wg3 = Wg3.astype(jnp.float32)

    src2 = src.reshape(-1, _CH)
    dst2 = dst.reshape(-1, _CH)

    h = _mlp(x, W1, b1, W2, b2)
    degp = _sc_degree(src2)

    t1p, dinv16 = _proj_scale(h, Wg1, degp)
    g1 = _sc_gather_scatter(t1p, src2, dst2)
    U = _hid_proj(g1, t1p, Wg2, Wg3, dinv16)
    g2 = _sc_gather_scatter(U, src2, dst2)
    mu, sc1 = _mu_kld(g2, U, dinv16)

    sall = _dense_loss(mu)
    zs, zd = _sc_edge_gather(mu, src2, dst2)
    sc2 = _edge_loss(zs, zd, src, dst)

    kld_sum = sc1[0, 0]
    diag_sp = sc1[0, 1]
    diag_p = sc1[0, 2]
    edge_sp = sc2[0, 0]
    edge_p = sc2[0, 1]
    all_sp = sall[0, 0]

    total = float(n) * float(n)
    s_edges = float(e)
    pos_weight = (total - s_edges) / s_edges
    norm = total / ((total - s_edges) * 2.0)

    bce = (all_sp
           + (pos_weight - 1.0) * (edge_sp + diag_sp)
           - pos_weight * (edge_p + diag_p)) / total
    kld = (-0.5) * kld_sum / total
    gae_loss = norm * bce + kld
    return h, gae_loss


# speedup vs baseline: 214.0650x; 1.2283x over previous
import functools

import jax
import jax.numpy as jnp
from jax import lax
from jax.experimental import pallas as pl
from jax.experimental.pallas import tpu as pltpu
from jax.experimental.pallas import tpu_sc as plsc

_N = 4096
_D = 128
_L = 64
_E = 65536

_NC = 2
_NS = 16
_NW = _NC * _NS
_CH = 128


def _softplus(x):
    return jnp.maximum(x, 0.0) + jnp.log1p(jnp.exp(-jnp.abs(x)))


def _scalar_tile(vals):
    r = lax.broadcasted_iota(jnp.int32, (8, 128), 0)
    c = lax.broadcasted_iota(jnp.int32, (8, 128), 1)
    out = jnp.zeros((8, 128), jnp.float32)
    for i, v in enumerate(vals):
        out = out + jnp.where((r == 0) & (c == i), v, 0.0)
    return out


def _dinv_block(degp_blk):
    deg = 1.0 + degp_blk[0][:, 0:1] + degp_blk[1][:, 0:1]
    return lax.rsqrt(deg)


def _mlp(x, W1, b1, W2, b2):
    n, d = x.shape
    BN = 512

    def body(xr, w1r, b1r, w2r, b2r, out):
        hb = jnp.maximum(
            jnp.dot(xr[...], w1r[...], preferred_element_type=jnp.float32, precision=lax.Precision.HIGHEST)
            + b1r[...], 0.0)
        out[...] = (
            jnp.dot(hb, w2r[...], preferred_element_type=jnp.float32, precision=lax.Precision.HIGHEST)
            + b2r[...])

    return pl.pallas_call(
        body,
        grid=(n // BN,),
        in_specs=[
            pl.BlockSpec((BN, d), lambda i: (i, 0)),
            pl.BlockSpec(W1.shape, lambda i: (0, 0)),
            pl.BlockSpec((1, b1.shape[0]), lambda i: (0, 0)),
            pl.BlockSpec(W2.shape, lambda i: (0, 0)),
            pl.BlockSpec((1, b2.shape[0]), lambda i: (0, 0)),
        ],
        out_specs=pl.BlockSpec((BN, W2.shape[1]), lambda i: (i, 0)),
        out_shape=jax.ShapeDtypeStruct((n, W2.shape[1]), jnp.float32),
    )(x, W1, b1.reshape(1, -1), W2, b2.reshape(1, -1))


def _proj_scale(h, Wg1, degp):
    n, l = h.shape
    BN = 512

    def body(hr, wr, dgr, out, dv_out):
        dinv = _dinv_block(dgr[...])
        t = jnp.dot(hr[...], wr[...], preferred_element_type=jnp.float32, precision=lax.Precision.HIGHEST) * dinv
        out[...] = jnp.concatenate(
            [t, jnp.zeros((BN, 128 - l), jnp.float32)], axis=1)
        dv_out[...] = jnp.broadcast_to(dinv, (BN, 16))

    return pl.pallas_call(
        body,
        grid=(n // BN,),
        in_specs=[
            pl.BlockSpec((BN, l), lambda i: (i, 0)),
            pl.BlockSpec(Wg1.shape, lambda i: (0, 0)),
            pl.BlockSpec((2, BN, 128), lambda i: (0, i, 0)),
        ],
        out_specs=[
            pl.BlockSpec((BN, 128), lambda i: (i, 0)),
            pl.BlockSpec((BN, 16), lambda i: (i, 0)),
        ],
        out_shape=[
            jax.ShapeDtypeStruct((n, 128), jnp.float32),
            jax.ShapeDtypeStruct((n, 16), jnp.float32),
        ],
    )(h, Wg1, degp)


def _hid_proj(g1, t1p, Wg2, Wg3, dinv16):
    n, _ = t1p.shape
    l = Wg2.shape[0]
    BN = 512

    def body(g1r, t1r, dvr, w2r, w3r, out):
        dinv = dvr[...][:, 0:1]
        gs = g1r[0] + g1r[1] + t1r[...]
        hid = jnp.maximum(dinv * gs[:, :l], 0.0)
        u2 = jnp.dot(hid, w2r[...], preferred_element_type=jnp.float32, precision=lax.Precision.HIGHEST) * dinv
        u3 = jnp.dot(hid, w3r[...], preferred_element_type=jnp.float32, precision=lax.Precision.HIGHEST) * dinv
        out[...] = jnp.concatenate([u2, u3], axis=1)

    return pl.pallas_call(
        body,
        grid=(n // BN,),
        in_specs=[
            pl.BlockSpec((2, BN, 128), lambda i: (0, i, 0)),
            pl.BlockSpec((BN, 128), lambda i: (i, 0)),
            pl.BlockSpec((BN, 16), lambda i: (i, 0)),
            pl.BlockSpec(Wg2.shape, lambda i: (0, 0)),
            pl.BlockSpec(Wg3.shape, lambda i: (0, 0)),
        ],
        out_specs=pl.BlockSpec((BN, 2 * l), lambda i: (i, 0)),
        out_shape=jax.ShapeDtypeStruct((n, 2 * l), jnp.float32),
    )(g1, t1p, dinv16, Wg2, Wg3)


def _mu_kld(g2, U, dinv16):
    n, d2 = U.shape
    l = d2 // 2
    BN = 512
    nblk = n // BN

    def body(g2r, ur, dvr, mu_out, sc_out, acc):
        i = pl.program_id(0)

        @pl.when(i == 0)
        def _():
            acc[0] = 0.0
            acc[1] = 0.0
            acc[2] = 0.0

        dinv = dvr[...][:, 0:1]
        v = dinv * (g2r[0] + g2r[1] + ur[...])
        mu = v[:, :l]
        lv = v[:, l:]
        mu_out[...] = jnp.concatenate(
            [mu, jnp.zeros((BN, d2 - l), jnp.float32)], axis=1)
        kt = jnp.sum(1.0 + 2.0 * lv - mu * mu - jnp.exp(2.0 * lv))
        q = jnp.sum(mu * mu, axis=1)
        acc[0] = acc[0] + kt
        acc[1] = acc[1] + jnp.sum(_softplus(q))
        acc[2] = acc[2] + jnp.sum(q)

        @pl.when(i == nblk - 1)
        def _():
            sc_out[...] = _scalar_tile([acc[0], acc[1], acc[2]])

    return pl.pallas_call(
        body,
        grid=(nblk,),
        in_specs=[
            pl.BlockSpec((2, BN, d2), lambda i: (0, i, 0)),
            pl.BlockSpec((BN, d2), lambda i: (i, 0)),
            pl.BlockSpec((BN, 16), lambda i: (i, 0)),
        ],
        out_specs=[
            pl.BlockSpec((BN, d2), lambda i: (i, 0)),
            pl.BlockSpec((8, 128), lambda i: (0, 0)),
        ],
        out_shape=[
            jax.ShapeDtypeStruct((n, d2), jnp.float32),
            jax.ShapeDtypeStruct((8, 128), jnp.float32),
        ],
        scratch_shapes=[pltpu.SMEM((4,), jnp.float32)],
    )(g2, U, dinv16)


def _dense_loss(mu):
    n, l = mu.shape
    BN = 512
    nblk = n // BN

    def body(ar, br, out, acc):
        i = pl.program_id(0)
        j = pl.program_id(1)

        @pl.when((i == 0) & (j == 0))
        def _():
            acc[0] = 0.0

        @pl.when(j >= i)
        def _():
            p = lax.dot_general(
                ar[...][:, :_L], br[...][:, :_L], (((1,), (1,)), ((), ())),
                preferred_element_type=jnp.float32,
                precision=lax.Precision.HIGHEST)
            w = jnp.where(j == i, 1.0, 2.0)
            acc[0] = acc[0] + w * jnp.sum(_softplus(p))

        @pl.when((i == nblk - 1) & (j == nblk - 1))
        def _():
            out[...] = _scalar_tile([acc[0]])

    return pl.pallas_call(
        body,
        grid=(nblk, nblk),
        in_specs=[
            pl.BlockSpec((BN, l), lambda i, j: (i, 0)),
            pl.BlockSpec((BN, l), lambda i, j: (j, 0)),
        ],
        out_specs=pl.BlockSpec((8, 128), lambda i, j: (0, 0)),
        out_shape=jax.ShapeDtypeStruct((8, 128), jnp.float32),
        scratch_shapes=[pltpu.SMEM((1,), jnp.float32)],
    )(mu, mu)


def _edge_loss(p, src, dst):
    e = p.shape[0]
    BE = 8192
    nblk = e // BE

    def body(pr, sr, dr, out, acc):
        i = pl.program_id(0)

        @pl.when(i == 0)
        def _():
            acc[0] = 0.0
            acc[1] = 0.0

        pv = pr[...]
        m = (sr[...] != dr[...]).astype(jnp.float32)
        acc[0] = acc[0] + jnp.sum(m * _softplus(pv))
        acc[1] = acc[1] + jnp.sum(m * pv)

        @pl.when(i == nblk - 1)
        def _():
            out[...] = _scalar_tile([acc[0], acc[1]])

    return pl.pallas_call(
        body,
        grid=(nblk,),
        in_specs=[
            pl.BlockSpec((BE,), lambda i: (i,)),
            pl.BlockSpec((BE,), lambda i: (i,)),
            pl.BlockSpec((BE,), lambda i: (i,)),
        ],
        out_specs=pl.BlockSpec((8, 128), lambda i: (0, 0)),
        out_shape=jax.ShapeDtypeStruct((8, 128), jnp.float32),
        scratch_shapes=[pltpu.SMEM((2,), jnp.float32)],
    )(p, src, dst)


def _sc_mesh():
    return plsc.VectorSubcoreMesh(core_axis_name="c", subcore_axis_name="s")


def _zero_rows(rows, d):
    z = jnp.zeros((16,), jnp.float32)

    def zb(i, carry):
        for j in range(d // 16):
            rows[i, pl.ds(j * 16, 16)] = z
        return carry

    lax.fori_loop(jnp.int32(0), jnp.int32(_CH), zb, jnp.int32(0))


def _sc_degree(src2):
    rows_per_tile = _N // _NS
    nch = _E // _NW // _CH

    @functools.partial(
        pl.kernel,
        out_type=jax.ShapeDtypeStruct((_NC, _N, 128), jnp.float32),
        mesh=_sc_mesh(),
        scratch_types=[
            pltpu.VMEM((nch, _CH), jnp.int32),
            pltpu.VMEM((_CH, 128), jnp.float32),
            pltpu.VMEM_SHARED((_N, 128), jnp.float32),
            pltpu.SemaphoreType.DMA,
        ],
    )
    def k(src_hbm, out_hbm, sidx, ones_v, acc_sh, sem):
        cid = lax.axis_index("c")
        sid = lax.axis_index("s")
        crow0 = (cid * _NS + sid) * nch
        pltpu.sync_copy(src_hbm.at[pl.ds(crow0, nch)], sidx)
        _zero_rows(ones_v, 128)
        for r in range(rows_per_tile // _CH):
            pltpu.sync_copy(
                ones_v, acc_sh.at[pl.ds(sid * rows_per_tile + r * _CH, _CH)])
        one = jnp.ones((16,), jnp.float32)

        def ob(i, carry):
            for j in range(128 // 16):
                ones_v[i, pl.ds(j * 16, 16)] = one
            return carry

        lax.fori_loop(jnp.int32(0), jnp.int32(_CH), ob, jnp.int32(0))
        plsc.subcore_barrier()
        for kk in range(nch):
            pltpu.sync_copy(ones_v, acc_sh.at[sidx.at[kk]], add=True)
        plsc.subcore_barrier()
        pltpu.sync_copy(
            acc_sh.at[pl.ds(sid * rows_per_tile, rows_per_tile)],
            out_hbm.at[cid, pl.ds(sid * rows_per_tile, rows_per_tile)])

    return k(src2)


def _sc_gather_scatter(s, src2, dst2):
    n, d = s.shape
    rows_per_tile = n // _NS
    nch = _E // _NW // _CH

    @functools.partial(
        pl.kernel,
        out_type=jax.ShapeDtypeStruct((_NC, n, d), jnp.float32),
        mesh=_sc_mesh(),
        scratch_types=[
            pltpu.VMEM((nch, _CH), jnp.int32),
            pltpu.VMEM((nch, _CH), jnp.int32),
            pltpu.VMEM((_CH, d), jnp.float32),
            pltpu.VMEM((_CH, d), jnp.float32),
            pltpu.VMEM_SHARED((n, d), jnp.float32),
            pltpu.SemaphoreType.DMA,
            pltpu.SemaphoreType.DMA,
        ],
    )
    def k(s_hbm, src_hbm, dst_hbm, out_hbm, sidx, didx, rows0, rows1,
          acc_sh, sem0, sem1):
        cid = lax.axis_index("c")
        sid = lax.axis_index("s")
        crow0 = (cid * _NS + sid) * nch
        pltpu.sync_copy(src_hbm.at[pl.ds(crow0, nch)], sidx)
        pltpu.sync_copy(dst_hbm.at[pl.ds(crow0, nch)], didx)
        _zero_rows(rows0, d)
        for r in range(rows_per_tile // _CH):
            pltpu.sync_copy(
                rows0, acc_sh.at[pl.ds(sid * rows_per_tile + r * _CH, _CH)])
        plsc.subcore_barrier()
        bufs = (rows0, rows1)
        sems = (sem0, sem1)
        pend = pltpu.async_copy(s_hbm.at[didx.at[0]], rows0, sem0)
        for kk in range(nch):
            pend.wait()
            if kk + 1 < nch:
                pend = pltpu.async_copy(
                    s_hbm.at[didx.at[kk + 1]],
                    bufs[(kk + 1) % 2], sems[(kk + 1) % 2])
            pltpu.sync_copy(bufs[kk % 2], acc_sh.at[sidx.at[kk]], add=True)
        plsc.subcore_barrier()
        pltpu.sync_copy(
            acc_sh.at[pl.ds(sid * rows_per_tile, rows_per_tile)],
            out_hbm.at[cid, pl.ds(sid * rows_per_tile, rows_per_tile)])

    return k(s, src2, dst2)


def _sc_edge_pred(mu, src2, dst2):
    n, d = mu.shape
    epw = _E // _NW
    nch = epw // _CH

    @functools.partial(
        pl.kernel,
        out_type=jax.ShapeDtypeStruct((_E,), jnp.float32),
        mesh=_sc_mesh(),
        compiler_params=pltpu.CompilerParams(needs_layout_passes=False),
        scratch_types=[
            pltpu.VMEM((nch, _CH), jnp.int32),
            pltpu.VMEM((nch, _CH), jnp.int32),
            pltpu.VMEM((_CH, d), jnp.float32),
            pltpu.VMEM((_CH, d), jnp.float32),
            pltpu.VMEM((_CH, d), jnp.float32),
            pltpu.VMEM((_CH, d), jnp.float32),
            pltpu.VMEM((epw,), jnp.float32),
            pltpu.SemaphoreType.DMA,
            pltpu.SemaphoreType.DMA,
            pltpu.SemaphoreType.DMA,
            pltpu.SemaphoreType.DMA,
        ],
    )
    def k(mu_hbm, src_hbm, dst_hbm, p_hbm, sidx, didx,
          bs0, bs1, bd0, bd1, p_tile, sem0, sem1, sem2, sem3):
        cid = lax.axis_index("c")
        sid = lax.axis_index("s")
        crow0 = (cid * _NS + sid) * nch
        base0 = (cid * _NS + sid) * epw
        pltpu.sync_copy(src_hbm.at[pl.ds(crow0, nch)], sidx)
        pltpu.sync_copy(dst_hbm.at[pl.ds(crow0, nch)], didx)
        sbufs = (bs0, bs1)
        dbufs = (bd0, bd1)
        ssems = (sem0, sem1)
        dsems = (sem2, sem3)
        lanes = lax.iota(jnp.int32, 16)

        def compute_chunk(kk, bs, bd):
            def grp(g, carry):
                row = lanes + g * jnp.int32(16)
                acc = jnp.zeros((16,), jnp.float32)
                for j in range(_L):
                    col = jnp.full((16,), j, jnp.int32)
                    a = plsc.load_gather(bs, [row, col])
                    b = plsc.load_gather(bd, [row, col])
                    acc = acc + a * b
                plsc.store_scatter(p_tile, [row + jnp.int32(kk * _CH)], acc)
                return carry

            lax.fori_loop(jnp.int32(0), jnp.int32(_CH // 16), grp,
                          jnp.int32(0))

        gs = [None] * nch
        gdd = [None] * nch
        gs[0] = pltpu.async_copy(mu_hbm.at[sidx.at[0]], bs0, sem0)
        gdd[0] = pltpu.async_copy(mu_hbm.at[didx.at[0]], bd0, sem2)
        for kk in range(nch):
            gs[kk].wait()
            gdd[kk].wait()
            if kk + 1 < nch:
                b = (kk + 1) % 2
                gs[kk + 1] = pltpu.async_copy(
                    mu_hbm.at[sidx.at[kk + 1]], sbufs[b], ssems[b])
                gdd[kk + 1] = pltpu.async_copy(
                    mu_hbm.at[didx.at[kk + 1]], dbufs[b], dsems[b])
            compute_chunk(kk, sbufs[kk % 2], dbufs[kk % 2])
        pltpu.sync_copy(p_tile, p_hbm.at[pl.ds(base0, epw)])

    return k(mu, src2, dst2)


def kernel(x, edge_index, W1, b1, W2, b2, Wg1, Wg2, Wg3):
    with jax.enable_x64(False):
        h, gae_loss = _run(x, edge_index, W1, b1, W2, b2, Wg1, Wg2, Wg3)
    return h.astype(jnp.float64), gae_loss.astype(jnp.float64)


def _run(x, edge_index, W1, b1, W2, b2, Wg1, Wg2, Wg3):
    n = x.shape[0]
    e = edge_index.shape[1]
    src = edge_index[0].astype(jnp.int32)
    dst = edge_index[1].astype(jnp.int32)
    x = x.astype(jnp.float32)
    W1 = W1.astype(jnp.float32)
    b1 = b1.astype(jnp.float32)
    W2 = W2.astype(jnp.float32)
    b2 = b2.astype(jnp.float32)
    Wg1 = Wg1.astype(jnp.float32)
    Wg2 = Wg2.astype(jnp.float32)
    Wg3 = Wg3.astype(jnp.float32)

    src2 = src.reshape(-1, _CH)
    dst2 = dst.reshape(-1, _CH)

    h = _mlp(x, W1, b1, W2, b2)
    degp = _sc_degree(src2)

    t1p, dinv16 = _proj_scale(h, Wg1, degp)
    g1 = _sc_gather_scatter(t1p, src2, dst2)
    U = _hid_proj(g1, t1p, Wg2, Wg3, dinv16)
    g2 = _sc_gather_scatter(U, src2, dst2)
    mu, sc1 = _mu_kld(g2, U, dinv16)

    sall = _dense_loss(mu)
    p_edges = _sc_edge_pred(mu, src2, dst2)
    sc2 = _edge_loss(p_edges, src, dst)

    kld_sum = sc1[0, 0]
    diag_sp = sc1[0, 1]
    diag_p = sc1[0, 2]
    edge_sp = sc2[0, 0]
    edge_p = sc2[0, 1]
    all_sp = sall[0, 0]

    total = float(n) * float(n)
    s_edges = float(e)
    pos_weight = (total - s_edges) / s_edges
    norm = total / ((total - s_edges) * 2.0)

    bce = (all_sp
           + (pos_weight - 1.0) * (edge_sp + diag_sp)
           - pos_weight * (edge_p + diag_p)) / total
    kld = (-0.5) * kld_sum / total
    gae_loss = norm * bce + kld
    return h, gae_loss


# speedup vs baseline: 244.1801x; 1.1407x over previous
import functools

import jax
import jax.numpy as jnp
from jax import lax
from jax.experimental import pallas as pl
from jax.experimental.pallas import tpu as pltpu
from jax.experimental.pallas import tpu_sc as plsc

_N = 4096
_D = 128
_L = 64
_E = 65536

_NC = 2
_NS = 16
_NW = _NC * _NS
_CH = 128


def _softplus(x):
    return jnp.maximum(x, 0.0) + jnp.log1p(jnp.exp(-jnp.abs(x)))


def _scalar_tile(vals):
    r = lax.broadcasted_iota(jnp.int32, (8, 128), 0)
    c = lax.broadcasted_iota(jnp.int32, (8, 128), 1)
    out = jnp.zeros((8, 128), jnp.float32)
    for i, v in enumerate(vals):
        out = out + jnp.where((r == 0) & (c == i), v, 0.0)
    return out


def _dinv_block(degp_blk):
    deg = 1.0 + degp_blk[0][:, 0:1] + degp_blk[1][:, 0:1]
    return lax.rsqrt(deg)


def _mlp(x, W1, b1, W2, b2):
    n, d = x.shape
    BN = 512

    def body(xr, w1r, b1r, w2r, b2r, out):
        hb = jnp.maximum(
            jnp.dot(xr[...], w1r[...], preferred_element_type=jnp.float32, precision=lax.Precision.HIGHEST)
            + b1r[...], 0.0)
        out[...] = (
            jnp.dot(hb, w2r[...], preferred_element_type=jnp.float32, precision=lax.Precision.HIGHEST)
            + b2r[...])

    return pl.pallas_call(
        body,
        grid=(n // BN,),
        in_specs=[
            pl.BlockSpec((BN, d), lambda i: (i, 0)),
            pl.BlockSpec(W1.shape, lambda i: (0, 0)),
            pl.BlockSpec((1, b1.shape[0]), lambda i: (0, 0)),
            pl.BlockSpec(W2.shape, lambda i: (0, 0)),
            pl.BlockSpec((1, b2.shape[0]), lambda i: (0, 0)),
        ],
        out_specs=pl.BlockSpec((BN, W2.shape[1]), lambda i: (i, 0)),
        out_shape=jax.ShapeDtypeStruct((n, W2.shape[1]), jnp.float32),
    )(x, W1, b1.reshape(1, -1), W2, b2.reshape(1, -1))


def _proj_scale(h, Wg1, degp):
    n, l = h.shape
    BN = 512

    def body(hr, wr, dgr, out, dv_out):
        dinv = _dinv_block(dgr[...])
        t = jnp.dot(hr[...], wr[...], preferred_element_type=jnp.float32, precision=lax.Precision.HIGHEST) * dinv
        out[...] = jnp.concatenate(
            [t, jnp.zeros((BN, 128 - l), jnp.float32)], axis=1)
        dv_out[...] = jnp.broadcast_to(dinv, (BN, 16))

    return pl.pallas_call(
        body,
        grid=(n // BN,),
        in_specs=[
            pl.BlockSpec((BN, l), lambda i: (i, 0)),
            pl.BlockSpec(Wg1.shape, lambda i: (0, 0)),
            pl.BlockSpec((2, BN, 128), lambda i: (0, i, 0)),
        ],
        out_specs=[
            pl.BlockSpec((BN, 128), lambda i: (i, 0)),
            pl.BlockSpec((BN, 16), lambda i: (i, 0)),
        ],
        out_shape=[
            jax.ShapeDtypeStruct((n, 128), jnp.float32),
            jax.ShapeDtypeStruct((n, 16), jnp.float32),
        ],
    )(h, Wg1, degp)


def _hid_proj(g1, t1p, Wg2, Wg3, dinv16):
    n, _ = t1p.shape
    l = Wg2.shape[0]
    BN = 512

    def body(g1r, t1r, dvr, w2r, w3r, out):
        dinv = dvr[...][:, 0:1]
        gs = g1r[0] + g1r[1] + t1r[...]
        hid = jnp.maximum(dinv * gs[:, :l], 0.0)
        u2 = jnp.dot(hid, w2r[...], preferred_element_type=jnp.float32, precision=lax.Precision.HIGHEST) * dinv
        u3 = jnp.dot(hid, w3r[...], preferred_element_type=jnp.float32, precision=lax.Precision.HIGHEST) * dinv
        out[...] = jnp.concatenate([u2, u3], axis=1)

    return pl.pallas_call(
        body,
        grid=(n // BN,),
        in_specs=[
            pl.BlockSpec((2, BN, 128), lambda i: (0, i, 0)),
            pl.BlockSpec((BN, 128), lambda i: (i, 0)),
            pl.BlockSpec((BN, 16), lambda i: (i, 0)),
            pl.BlockSpec(Wg2.shape, lambda i: (0, 0)),
            pl.BlockSpec(Wg3.shape, lambda i: (0, 0)),
        ],
        out_specs=pl.BlockSpec((BN, 2 * l), lambda i: (i, 0)),
        out_shape=jax.ShapeDtypeStruct((n, 2 * l), jnp.float32),
    )(g1, t1p, dinv16, Wg2, Wg3)


def _mu_kld(g2, U, dinv16):
    n, d2 = U.shape
    l = d2 // 2
    BN = 512
    nblk = n // BN

    def body(g2r, ur, dvr, mu_out, sc_out, acc):
        i = pl.program_id(0)

        @pl.when(i == 0)
        def _():
            acc[0] = 0.0
            acc[1] = 0.0
            acc[2] = 0.0

        dinv = dvr[...][:, 0:1]
        v = dinv * (g2r[0] + g2r[1] + ur[...])
        mu = v[:, :l]
        lv = v[:, l:]
        mu_out[...] = jnp.concatenate(
            [mu, jnp.zeros((BN, d2 - l), jnp.float32)], axis=1)
        kt = jnp.sum(1.0 + 2.0 * lv - mu * mu - jnp.exp(2.0 * lv))
        q = jnp.sum(mu * mu, axis=1)
        acc[0] = acc[0] + kt
        acc[1] = acc[1] + jnp.sum(_softplus(q))
        acc[2] = acc[2] + jnp.sum(q)

        @pl.when(i == nblk - 1)
        def _():
            sc_out[...] = _scalar_tile([acc[0], acc[1], acc[2]])

    return pl.pallas_call(
        body,
        grid=(nblk,),
        in_specs=[
            pl.BlockSpec((2, BN, d2), lambda i: (0, i, 0)),
            pl.BlockSpec((BN, d2), lambda i: (i, 0)),
            pl.BlockSpec((BN, 16), lambda i: (i, 0)),
        ],
        out_specs=[
            pl.BlockSpec((BN, d2), lambda i: (i, 0)),
            pl.BlockSpec((8, 128), lambda i: (0, 0)),
        ],
        out_shape=[
            jax.ShapeDtypeStruct((n, d2), jnp.float32),
            jax.ShapeDtypeStruct((8, 128), jnp.float32),
        ],
        scratch_shapes=[pltpu.SMEM((4,), jnp.float32)],
    )(g2, U, dinv16)


def _dense_loss(mu):
    n, l = mu.shape
    BN = 512
    nblk = n // BN

    def body(ar, br, out, acc):
        i = pl.program_id(0)
        j = pl.program_id(1)

        @pl.when((i == 0) & (j == 0))
        def _():
            acc[0] = 0.0

        @pl.when(j >= i)
        def _():
            p = lax.dot_general(
                ar[...][:, :_L], br[...][:, :_L], (((1,), (1,)), ((), ())),
                preferred_element_type=jnp.float32,
                precision=lax.Precision.HIGHEST)
            w = jnp.where(j == i, 1.0, 2.0)
            acc[0] = acc[0] + w * jnp.sum(_softplus(p))

        @pl.when((i == nblk - 1) & (j == nblk - 1))
        def _():
            out[...] = _scalar_tile([acc[0]])

    return pl.pallas_call(
        body,
        grid=(nblk, nblk),
        in_specs=[
            pl.BlockSpec((BN, l), lambda i, j: (i, 0)),
            pl.BlockSpec((BN, l), lambda i, j: (j, 0)),
        ],
        out_specs=pl.BlockSpec((8, 128), lambda i, j: (0, 0)),
        out_shape=jax.ShapeDtypeStruct((8, 128), jnp.float32),
        scratch_shapes=[pltpu.SMEM((1,), jnp.float32)],
    )(mu, mu)


def _edge_loss(p, src, dst):
    e = p.shape[0]
    BE = 8192
    nblk = e // BE

    def body(pr, sr, dr, out, acc):
        i = pl.program_id(0)

        @pl.when(i == 0)
        def _():
            acc[0] = 0.0
            acc[1] = 0.0

        pv = pr[...]
        m = (sr[...] != dr[...]).astype(jnp.float32)
        acc[0] = acc[0] + jnp.sum(m * _softplus(pv))
        acc[1] = acc[1] + jnp.sum(m * pv)

        @pl.when(i == nblk - 1)
        def _():
            out[...] = _scalar_tile([acc[0], acc[1]])

    return pl.pallas_call(
        body,
        grid=(nblk,),
        in_specs=[
            pl.BlockSpec((BE,), lambda i: (i,)),
            pl.BlockSpec((BE,), lambda i: (i,)),
            pl.BlockSpec((BE,), lambda i: (i,)),
        ],
        out_specs=pl.BlockSpec((8, 128), lambda i: (0, 0)),
        out_shape=jax.ShapeDtypeStruct((8, 128), jnp.float32),
        scratch_shapes=[pltpu.SMEM((2,), jnp.float32)],
    )(p, src, dst)


def _sc_mesh():
    return plsc.VectorSubcoreMesh(core_axis_name="c", subcore_axis_name="s")


def _zero_rows(rows, d):
    z = jnp.zeros((16,), jnp.float32)

    def zb(i, carry):
        for j in range(d // 16):
            rows[i, pl.ds(j * 16, 16)] = z
        return carry

    lax.fori_loop(jnp.int32(0), jnp.int32(_CH), zb, jnp.int32(0))


def _sc_degree(src2):
    rows_per_tile = _N // _NS
    nch = _E // _NW // _CH

    @functools.partial(
        pl.kernel,
        out_type=jax.ShapeDtypeStruct((_NC, _N, 128), jnp.float32),
        mesh=_sc_mesh(),
        scratch_types=[
            pltpu.VMEM((nch, _CH), jnp.int32),
            pltpu.VMEM((_CH, 128), jnp.float32),
            pltpu.VMEM_SHARED((_N, 128), jnp.float32),
            pltpu.SemaphoreType.DMA,
        ],
    )
    def k(src_hbm, out_hbm, sidx, ones_v, acc_sh, sem):
        cid = lax.axis_index("c")
        sid = lax.axis_index("s")
        crow0 = (cid * _NS + sid) * nch
        pltpu.sync_copy(src_hbm.at[pl.ds(crow0, nch)], sidx)
        _zero_rows(ones_v, 128)
        for r in range(rows_per_tile // _CH):
            pltpu.sync_copy(
                ones_v, acc_sh.at[pl.ds(sid * rows_per_tile + r * _CH, _CH)])
        one = jnp.ones((16,), jnp.float32)

        def ob(i, carry):
            for j in range(128 // 16):
                ones_v[i, pl.ds(j * 16, 16)] = one
            return carry

        lax.fori_loop(jnp.int32(0), jnp.int32(_CH), ob, jnp.int32(0))
        plsc.subcore_barrier()
        for kk in range(nch):
            pltpu.sync_copy(ones_v, acc_sh.at[sidx.at[kk]], add=True)
        plsc.subcore_barrier()
        pltpu.sync_copy(
            acc_sh.at[pl.ds(sid * rows_per_tile, rows_per_tile)],
            out_hbm.at[cid, pl.ds(sid * rows_per_tile, rows_per_tile)])

    return k(src2)


def _sc_gather_scatter(s, src2, dst2):
    n, d = s.shape
    rows_per_tile = n // _NS
    nch = _E // _NW // _CH

    @functools.partial(
        pl.kernel,
        out_type=jax.ShapeDtypeStruct((_NC, n, d), jnp.float32),
        mesh=_sc_mesh(),
        scratch_types=[
            pltpu.VMEM((nch, _CH), jnp.int32),
            pltpu.VMEM((nch, _CH), jnp.int32),
            pltpu.VMEM((_CH, d), jnp.float32),
            pltpu.VMEM((_CH, d), jnp.float32),
            pltpu.VMEM_SHARED((n, d), jnp.float32),
            pltpu.SemaphoreType.DMA,
            pltpu.SemaphoreType.DMA,
        ],
    )
    def k(s_hbm, src_hbm, dst_hbm, out_hbm, sidx, didx, rows0, rows1,
          acc_sh, sem0, sem1):
        cid = lax.axis_index("c")
        sid = lax.axis_index("s")
        crow0 = (cid * _NS + sid) * nch
        pltpu.sync_copy(src_hbm.at[pl.ds(crow0, nch)], sidx)
        pltpu.sync_copy(dst_hbm.at[pl.ds(crow0, nch)], didx)
        _zero_rows(rows0, d)
        for r in range(rows_per_tile // _CH):
            pltpu.sync_copy(
                rows0, acc_sh.at[pl.ds(sid * rows_per_tile + r * _CH, _CH)])
        plsc.subcore_barrier()
        bufs = (rows0, rows1)
        sems = (sem0, sem1)
        pend = pltpu.async_copy(s_hbm.at[didx.at[0]], rows0, sem0)
        for kk in range(nch):
            pend.wait()
            if kk + 1 < nch:
                pend = pltpu.async_copy(
                    s_hbm.at[didx.at[kk + 1]],
                    bufs[(kk + 1) % 2], sems[(kk + 1) % 2])
            pltpu.sync_copy(bufs[kk % 2], acc_sh.at[sidx.at[kk]], add=True)
        plsc.subcore_barrier()
        pltpu.sync_copy(
            acc_sh.at[pl.ds(sid * rows_per_tile, rows_per_tile)],
            out_hbm.at[cid, pl.ds(sid * rows_per_tile, rows_per_tile)])

    return k(s, src2, dst2)


def _sc_edge_pred(mu, src2, dst2):
    n, d = mu.shape
    epw = _E // _NW
    nch = epw // _CH

    @functools.partial(
        pl.kernel,
        out_type=jax.ShapeDtypeStruct((_E,), jnp.float32),
        mesh=_sc_mesh(),
        compiler_params=pltpu.CompilerParams(needs_layout_passes=False),
        scratch_types=[
            pltpu.VMEM((nch, _CH), jnp.int32),
            pltpu.VMEM((nch, _CH), jnp.int32),
            pltpu.VMEM((_CH, d), jnp.float32),
            pltpu.VMEM((_CH, d), jnp.float32),
            pltpu.VMEM((_CH, d), jnp.float32),
            pltpu.VMEM((_CH, d), jnp.float32),
            pltpu.VMEM((epw,), jnp.float32),
            pltpu.SemaphoreType.DMA,
            pltpu.SemaphoreType.DMA,
            pltpu.SemaphoreType.DMA,
            pltpu.SemaphoreType.DMA,
        ],
    )
    def k(mu_hbm, src_hbm, dst_hbm, p_hbm, sidx, didx,
          bs0, bs1, bd0, bd1, p_tile, sem0, sem1, sem2, sem3):
        cid = lax.axis_index("c")
        sid = lax.axis_index("s")
        crow0 = (cid * _NS + sid) * nch
        base0 = (cid * _NS + sid) * epw
        pltpu.sync_copy(src_hbm.at[pl.ds(crow0, nch)], sidx)
        pltpu.sync_copy(dst_hbm.at[pl.ds(crow0, nch)], didx)
        sbufs = (bs0, bs1)
        dbufs = (bd0, bd1)
        ssems = (sem0, sem1)
        dsems = (sem2, sem3)
        lanes = lax.iota(jnp.int32, 16)

        def compute_chunk(kk, bs, bd):
            def grp(g, carry):
                row = lanes + g * jnp.int32(16)

                def jblk(jj, acc):
                    j0 = jj * jnp.int32(8)
                    for u in range(8):
                        col = jnp.bitwise_and(lanes + (j0 + jnp.int32(u)),
                                              jnp.int32(_L - 1))
                        a = plsc.load_gather(bs, [row, col])
                        b = plsc.load_gather(bd, [row, col])
                        acc = acc + a * b
                    return acc

                acc = lax.fori_loop(jnp.int32(0), jnp.int32(_L // 8), jblk,
                                    jnp.zeros((16,), jnp.float32))
                plsc.store_scatter(p_tile, [row + jnp.int32(kk * _CH)], acc)
                return carry

            lax.fori_loop(jnp.int32(0), jnp.int32(_CH // 16), grp,
                          jnp.int32(0))

        gs = [None] * nch
        gdd = [None] * nch
        gs[0] = pltpu.async_copy(mu_hbm.at[sidx.at[0]], bs0, sem0)
        gdd[0] = pltpu.async_copy(mu_hbm.at[didx.at[0]], bd0, sem2)
        for kk in range(nch):
            gs[kk].wait()
            gdd[kk].wait()
            if kk + 1 < nch:
                b = (kk + 1) % 2
                gs[kk + 1] = pltpu.async_copy(
                    mu_hbm.at[sidx.at[kk + 1]], sbufs[b], ssems[b])
                gdd[kk + 1] = pltpu.async_copy(
                    mu_hbm.at[didx.at[kk + 1]], dbufs[b], dsems[b])
            compute_chunk(kk, sbufs[kk % 2], dbufs[kk % 2])
        pltpu.sync_copy(p_tile, p_hbm.at[pl.ds(base0, epw)])

    return k(mu, src2, dst2)


def kernel(x, edge_index, W1, b1, W2, b2, Wg1, Wg2, Wg3):
    with jax.enable_x64(False):
        h, gae_loss = _run(x, edge_index, W1, b1, W2, b2, Wg1, Wg2, Wg3)
    return h.astype(jnp.float64), gae_loss.astype(jnp.float64)


def _run(x, edge_index, W1, b1, W2, b2, Wg1, Wg2, Wg3):
    n = x.shape[0]
    e = edge_index.shape[1]
    src = edge_index[0].astype(jnp.int32)
    dst = edge_index[1].astype(jnp.int32)
    x = x.astype(jnp.float32)
    W1 = W1.astype(jnp.float32)
    b1 = b1.astype(jnp.float32)
    W2 = W2.astype(jnp.float32)
    b2 = b2.astype(jnp.float32)
    Wg1 = Wg1.astype(jnp.float32)
    Wg2 = Wg2.astype(jnp.float32)
    Wg3 = Wg3.astype(jnp.float32)

    src2 = src.reshape(-1, _CH)
    dst2 = dst.reshape(-1, _CH)

    h = _mlp(x, W1, b1, W2, b2)
    degp = _sc_degree(src2)

    t1p, dinv16 = _proj_scale(h, Wg1, degp)
    g1 = _sc_gather_scatter(t1p, src2, dst2)
    U = _hid_proj(g1, t1p, Wg2, Wg3, dinv16)
    g2 = _sc_gather_scatter(U, src2, dst2)
    mu, sc1 = _mu_kld(g2, U, dinv16)

    sall = _dense_loss(mu)
    p_edges = _sc_edge_pred(mu, src2, dst2)
    sc2 = _edge_loss(p_edges, src, dst)

    kld_sum = sc1[0, 0]
    diag_sp = sc1[0, 1]
    diag_p = sc1[0, 2]
    edge_sp = sc2[0, 0]
    edge_p = sc2[0, 1]
    all_sp = sall[0, 0]

    total = float(n) * float(n)
    s_edges = float(e)
    pos_weight = (total - s_edges) / s_edges
    norm = total / ((total - s_edges) * 2.0)

    bce = (all_sp
           + (pos_weight - 1.0) * (edge_sp + diag_sp)
           - pos_weight * (edge_p + diag_p)) / total
    kld = (-0.5) * kld_sum / total
    gae_loss = norm * bce + kld
    return h, gae_loss
